# Initial kernel scaffold; baseline (speedup 1.0000x reference)
import jax
import jax.numpy as jnp
from jax.experimental import pallas as pl


def kernel(x, edge_index, edge_weight, W1, b1, W2, b2, W_ih, W_hh, b_ih, b_hh, heads_W, heads_b):
    raise NotImplementedError("write your pallas kernel here")


# speedup vs baseline: 10.5160x; 10.5160x over previous
import jax
import jax.numpy as jnp
from jax import lax
from jax.experimental import pallas as pl
from jax.experimental.pallas import tpu as pltpu
from jax.experimental.pallas import tpu_sc as plsc

_T, _N, _D, _H = 12, 10000, 128, 64
_E = 320000
_NT = 16
_NPT = 640
_NPAD = _NT * _NPT
_EPT = _E // _NT
_CH = 1000
_NCH = _EPT // _CH
_STG = 1088
_ROW = 22528
_DUMP = _NPAD
_AGG_ROWS = _NPAD + 8


def _graph_body(s_hbm, d_hbm, ew_hbm, x_hbm, w1_hbm, b1_hbm,
                up_hbm, degp_hbm, c0p_hbm, selS_hbm, selT_hbm, selW_hbm,
                s_ch, d_ch, ew_ch,
                nodef_a, nodef_b,
                degsl, c0sl, invsl, w0cmp, zrow, tmp,
                stg_s, stg_t, stg_w,
                w1_loc, b1_loc,
                gidx, sidx, xrow, arow, u_loc,
                dinv_sh, inv_sh, agg_sh,
                sem):
    tid = lax.axis_index("s")
    i16 = lax.iota(jnp.int32, 16)
    zf = jnp.zeros((16,), jnp.float32)
    nb = tid * _NPT

    def z_big(i, c):
        nodef_a[pl.ds(i * 16, 16)] = zf
        nodef_b[pl.ds(i * 16, 16)] = zf
        return c
    lax.fori_loop(0, _NPAD // 16, z_big, 0)

    def z_small(i, c):
        degsl[pl.ds(i * 16, 16)] = zf
        c0sl[pl.ds(i * 16, 16)] = zf
        return c
    lax.fori_loop(0, _NPT // 16, z_small, 0)

    def z_u(i, c):
        u_loc[pl.ds(i * 16, 16)] = zf
        return c
    lax.fori_loop(0, (_T * _H) // 16, z_u, 0)
    zrow[pl.ds(0, 16)] = zf
    for q in range(1, _D // 16):
        zrow[pl.ds(q * 16, 16)] = zf

    pltpu.sync_copy(w1_hbm, w1_loc)
    pltpu.sync_copy(b1_hbm, b1_loc)

    def p1_chunk(c, carry):
        base = pl.multiple_of(tid * _EPT + c * _CH, 8)
        pltpu.sync_copy(s_hbm.at[pl.ds(base, _CH)], s_ch)
        pltpu.sync_copy(d_hbm.at[pl.ds(base, _CH)], d_ch)
        pltpu.sync_copy(ew_hbm.at[pl.ds(base, _CH)], ew_ch)

        def p1_v(i, cc):
            sv = s_ch[pl.ds(i * 16, 16)]
            dv = d_ch[pl.ds(i * 16, 16)]
            ev = ew_ch[pl.ds(i * 16, 16)]
            plsc.addupdate_scatter(nodef_a, [dv], ev)
            plsc.addupdate_scatter(nodef_b, [sv],
                                   jnp.where(dv == 0, ev, jnp.float32(0.0)))
            return cc
        lax.fori_loop(0, _CH // 16, p1_v, 0)
        return carry
    lax.fori_loop(0, _NCH, p1_chunk, 0)

    pltpu.sync_copy(nodef_a, degp_hbm.at[pl.ds(pl.multiple_of(tid * _NPAD, 8), _NPAD)])
    pltpu.sync_copy(nodef_b, c0p_hbm.at[pl.ds(pl.multiple_of(tid * _NPAD, 8), _NPAD)])
    plsc.subcore_barrier()
    for j in range(_NT):
        pltpu.sync_copy(degp_hbm.at[pl.ds(pl.multiple_of(j * _NPAD + nb, 8), _NPT)], tmp)

        def accd(i, c):
            degsl[pl.ds(i * 16, 16)] = degsl[pl.ds(i * 16, 16)] + tmp[pl.ds(i * 16, 16)]
            return c
        lax.fori_loop(0, _NPT // 16, accd, 0)
        pltpu.sync_copy(c0p_hbm.at[pl.ds(pl.multiple_of(j * _NPAD + nb, 8), _NPT)], tmp)

        def accc(i, c):
            c0sl[pl.ds(i * 16, 16)] = c0sl[pl.ds(i * 16, 16)] + tmp[pl.ds(i * 16, 16)]
            return c
        lax.fori_loop(0, _NPT // 16, accc, 0)

    def p3(i, c):
        dg = degsl[pl.ds(i * 16, 16)] + 1.0
        ib = plsc.bitcast(dg, jnp.int32)
        ib = 0x5F3759DF - (ib >> 1)
        y = plsc.bitcast(ib, jnp.float32)
        y = y * (1.5 - 0.5 * dg * y * y)
        y = y * (1.5 - 0.5 * dg * y * y)
        y = y * (1.5 - 0.5 * dg * y * y)
        degsl[pl.ds(i * 16, 16)] = y
        return c
    lax.fori_loop(0, _NPT // 16, p3, 0)
    pltpu.sync_copy(degsl, dinv_sh.at[pl.ds(pl.multiple_of(nb, 8), _NPT)])
    plsc.subcore_barrier()
    pltpu.sync_copy(dinv_sh, nodef_a)

    def _flush(off, cnt):
        for p in range(4):
            pos = cnt + p * 16 + i16
            plsc.store_scatter(stg_s, [pos], jnp.zeros((16,), jnp.int32))
            plsc.store_scatter(stg_t, [pos], jnp.full((16,), _DUMP, jnp.int32))
            plsc.store_scatter(stg_w, [pos], zf)
        padded = ((cnt + 63) >> 6) << 6

        def fl(p, c):
            dst = pl.multiple_of(tid * _ROW + off + p * 64, 8)
            pltpu.sync_copy(stg_s.at[pl.ds(p * 64, 64)],
                            selS_hbm.at[pl.ds(dst, 64)])
            pltpu.sync_copy(stg_t.at[pl.ds(p * 64, 64)],
                            selT_hbm.at[pl.ds(dst, 64)])
            pltpu.sync_copy(stg_w.at[pl.ds(p * 64, 64)],
                            selW_hbm.at[pl.ds(dst, 64)])
            return c
        lax.fori_loop(0, padded >> 6, fl, 0)
        return off + padded

    dinv0 = nodef_a[pl.ds(0, 16)][0]

    def p4(i, st):
        lslot = st
        gid = nb + i * 16 + i16
        c0v = c0sl[pl.ds(i * 16, 16)]
        dvv = degsl[pl.ds(i * 16, 16)]
        m = (c0v > 0.0) | (gid == 0)
        mi = m.astype(jnp.int32)
        excl = plsc.cumsum(mi) - mi
        cnt = jnp.sum(mi)
        slot_local = lslot + excl
        w0v = dinv0 * (c0v * dvv + jnp.where(gid == 0, dinv0, jnp.float32(0.0)))
        plsc.store_scatter(w0cmp, [slot_local], w0v, mask=m)
        invsl[pl.ds(i * 16, 16)] = jnp.where(
            m, (nb + slot_local + 1).astype(jnp.float32), jnp.float32(0.0))
        plsc.store_scatter(stg_s, [slot_local], gid, mask=m)
        plsc.store_scatter(stg_t, [slot_local], nb + slot_local, mask=m)
        plsc.store_scatter(stg_w, [slot_local], dvv * dvv, mask=m)
        return lslot + cnt
    cnt_own = lax.fori_loop(0, _NPT // 16, p4, jnp.int32(0))
    off = _flush(jnp.int32(0), cnt_own)
    pltpu.sync_copy(invsl, inv_sh.at[pl.ds(pl.multiple_of(nb, 8), _NPT)])
    plsc.subcore_barrier()
    pltpu.sync_copy(inv_sh, nodef_b)

    def p5_chunk(c, off):
        base = pl.multiple_of(tid * _EPT + c * _CH, 8)
        pltpu.sync_copy(s_hbm.at[pl.ds(base, _CH)], s_ch)
        pltpu.sync_copy(d_hbm.at[pl.ds(base, _CH)], d_ch)
        pltpu.sync_copy(ew_hbm.at[pl.ds(base, _CH)], ew_ch)

        def p5_v(i, lc):
            sv = s_ch[pl.ds(i * 16, 16)]
            dv = d_ch[pl.ds(i * 16, 16)]
            ev = ew_ch[pl.ds(i * 16, 16)]
            g = plsc.load_gather(nodef_b, [dv])
            m = g > 0.5
            mi = m.astype(jnp.int32)
            app = lc + plsc.cumsum(mi) - mi
            nrm = (plsc.load_gather(nodef_a, [sv]) * ev *
                   plsc.load_gather(nodef_a, [dv]))
            plsc.store_scatter(stg_s, [app], sv, mask=m)
            plsc.store_scatter(stg_t, [app], (g - 1.0).astype(jnp.int32),
                               mask=m)
            plsc.store_scatter(stg_w, [app], nrm, mask=m)
            return lc + jnp.sum(mi)
        lc = lax.fori_loop(0, _CH // 16, p5_v, jnp.int32(0))
        return _flush(off, lc)
    off = lax.fori_loop(0, _NCH, p5_chunk, off)
    ngroups = off >> 4

    def t_loop(t, carry):
        def za(j, c):
            pltpu.sync_copy(zrow, agg_sh.at[nb + j])
            return c
        lax.fori_loop(0, cnt_own, za, 0)
        plsc.subcore_barrier()

        def cchunk(ci, c):
            cbase = pl.multiple_of(tid * _ROW + ci * 1024, 8)
            pltpu.sync_copy(selS_hbm.at[pl.ds(cbase, 1024)],
                            stg_s.at[pl.ds(0, 1024)])
            pltpu.sync_copy(selT_hbm.at[pl.ds(cbase, 1024)],
                            stg_t.at[pl.ds(0, 1024)])
            pltpu.sync_copy(selW_hbm.at[pl.ds(cbase, 1024)],
                            stg_w.at[pl.ds(0, 1024)])
            ng_here = jnp.minimum(64, ngroups - ci * 64)

            def gb(gl, cc):
                gidx[...] = stg_s[pl.ds(gl * 16, 16)] + t * _N
                sidx[...] = stg_t[pl.ds(gl * 16, 16)]
                nv = stg_w[pl.ds(gl * 16, 16)]
                pltpu.async_copy(x_hbm.at[gidx], xrow, sem).wait()
                for r in range(16):
                    nrm = nv[r]
                    for q in range(_D // 16):
                        xrow[r, pl.ds(q * 16, 16)] = (
                            xrow[r, pl.ds(q * 16, 16)] * nrm)
                pltpu.sync_copy(xrow, agg_sh.at[sidx], add=True)
                return cc
            lax.fori_loop(0, ng_here, gb, 0)
            return c
        lax.fori_loop(0, (off + 1023) >> 10, cchunk, 0)
        plsc.subcore_barrier()

        def cs(j, c):
            pltpu.sync_copy(agg_sh.at[nb + j], arow)
            w0j = w0cmp[pl.ds(j, 16)][0]

            def kk(kb, acc):
                a0, a1, a2, a3 = acc
                av = arow[pl.ds(kb * 16, 16)]
                for r in range(16):
                    ak = av[r]
                    k = kb * 16 + r
                    a0 = a0 + ak * w1_loc[pl.ds(k * 64, 16)]
                    a1 = a1 + ak * w1_loc[pl.ds(k * 64 + 16, 16)]
                    a2 = a2 + ak * w1_loc[pl.ds(k * 64 + 32, 16)]
                    a3 = a3 + ak * w1_loc[pl.ds(k * 64 + 48, 16)]
                return (a0, a1, a2, a3)
            accs = lax.fori_loop(0, _D // 16, kk, (zf, zf, zf, zf))
            for q in range(4):
                hq = jnp.maximum(accs[q] + b1_loc[pl.ds(q * 16, 16)], 0.0)
                u_loc[pl.ds(t * 64 + q * 16, 16)] = (
                    u_loc[pl.ds(t * 64 + q * 16, 16)] + w0j * hq)
            return c
        lax.fori_loop(0, cnt_own, cs, 0)
        return carry
    lax.fori_loop(0, _T, t_loop, 0)

    pltpu.sync_copy(u_loc, up_hbm.at[pl.ds(pl.multiple_of(tid * _T * _H, 8), _T * _H)])


def _graph_call(s, d, ew, x2d, w1f, b1):
    return pl.kernel(
        _graph_body,
        out_type=(
            jax.ShapeDtypeStruct((_NT * _T * _H,), jnp.float32),
            jax.ShapeDtypeStruct((_NT * _NPAD,), jnp.float32),
            jax.ShapeDtypeStruct((_NT * _NPAD,), jnp.float32),
            jax.ShapeDtypeStruct((_NT * _ROW,), jnp.int32),
            jax.ShapeDtypeStruct((_NT * _ROW,), jnp.int32),
            jax.ShapeDtypeStruct((_NT * _ROW,), jnp.float32),
        ),
        mesh=plsc.VectorSubcoreMesh(core_axis_name="c", subcore_axis_name="s",
                                    num_cores=1),
        compiler_params=pltpu.CompilerParams(needs_layout_passes=False),
        scratch_types=[
            pltpu.VMEM((_CH,), jnp.int32),
            pltpu.VMEM((_CH,), jnp.int32),
            pltpu.VMEM((_CH,), jnp.float32),
            pltpu.VMEM((_NPAD,), jnp.float32),
            pltpu.VMEM((_NPAD,), jnp.float32),
            pltpu.VMEM((_NPT,), jnp.float32),
            pltpu.VMEM((_NPT,), jnp.float32),
            pltpu.VMEM((_NPT,), jnp.float32),
            pltpu.VMEM((_NPT + 16,), jnp.float32),
            pltpu.VMEM((_D,), jnp.float32),
            pltpu.VMEM((_NPT,), jnp.float32),
            pltpu.VMEM((_STG,), jnp.int32),
            pltpu.VMEM((_STG,), jnp.int32),
            pltpu.VMEM((_STG,), jnp.float32),
            pltpu.VMEM((_D * _H,), jnp.float32),
            pltpu.VMEM((_H,), jnp.float32),
            pltpu.VMEM((16,), jnp.int32),
            pltpu.VMEM((16,), jnp.int32),
            pltpu.VMEM((16, _D), jnp.float32),
            pltpu.VMEM((_D,), jnp.float32),
            pltpu.VMEM((_T * _H,), jnp.float32),
            pltpu.VMEM_SHARED((_NPAD,), jnp.float32),
            pltpu.VMEM_SHARED((_NPAD,), jnp.float32),
            pltpu.VMEM_SHARED((_AGG_ROWS, _D), jnp.float32),
            pltpu.SemaphoreType.DMA,
        ],
    )(s, d, ew, x2d, w1f, b1)


def _head_body(m_ref, up_ref, w2_ref, b2_ref, wih_ref, whh_ref,
               bih_ref, bhh_ref, hwt_ref, hb_ref, out_ref):
    u = jnp.dot(m_ref[...], up_ref[...], preferred_element_type=jnp.float32)
    seq = jnp.maximum(
        jnp.dot(u, w2_ref[...], preferred_element_type=jnp.float32)
        + b2_ref[...][None, :], 0.0)
    h = jnp.zeros((1, _H), jnp.float32)
    for t in range(_T):
        xt = seq[t:t + 1, :]
        gi = jnp.dot(xt, wih_ref[...],
                     preferred_element_type=jnp.float32) + bih_ref[...][None, :]
        gh = jnp.dot(h, whh_ref[...],
                     preferred_element_type=jnp.float32) + bhh_ref[...][None, :]
        r = jax.nn.sigmoid(gi[:, :_H] + gh[:, :_H])
        z = jax.nn.sigmoid(gi[:, _H:2 * _H] + gh[:, _H:2 * _H])
        n = jnp.tanh(gi[:, 2 * _H:] + r * gh[:, 2 * _H:])
        h = (1.0 - z) * n + z * h
    logits = jnp.dot(h, hwt_ref[...],
                     preferred_element_type=jnp.float32) + hb_ref[...][None, :]
    out_ref[...] = jax.nn.sigmoid(logits)


def _head_call(m, up, W2, b2, wihT, whhT, b_ih, b_hh, hWT, heads_b):
    return pl.pallas_call(
        _head_body,
        out_shape=jax.ShapeDtypeStruct((1, 8), jnp.float32),
    )(m, up, W2, b2, wihT, whhT, b_ih, b_hh, hWT, heads_b)


def kernel(x, edge_index, edge_weight, W1, b1, W2, b2, W_ih, W_hh,
           b_ih, b_hh, heads_W, heads_b):
    s = edge_index[0].astype(jnp.int32)
    d = edge_index[1].astype(jnp.int32)
    x2d = x.reshape(_T * _N, _D)
    outs = _graph_call(s, d, edge_weight.astype(jnp.float32), x2d,
                       W1.reshape(-1), b1)
    up = outs[0].reshape(_NT * _T, _H)
    m = jnp.tile(jnp.eye(_T, dtype=jnp.float32), (1, _NT))
    return _head_call(m, up, W2, b2, W_ih.T, W_hh.T, b_ih, b_hh,
                      heads_W.T, heads_b)


# speedup vs baseline: 244.8001x; 23.2788x over previous
import jax
import jax.numpy as jnp
from jax import lax
from jax.experimental import pallas as pl
from jax.experimental.pallas import tpu as pltpu
from jax.experimental.pallas import tpu_sc as plsc

_T, _N, _D, _H = 12, 10000, 128, 64
_E = 320000
_NT = 16
_NPT = 640
_NPAD = _NT * _NPT
_EPT = _E // _NT
_CH = 1000
_NCH = _EPT // _CH
_STG = 2112
_ROW = 22528
_DUMP = _NPAD
_AGG_ROWS = _NPAD + 8


def _graph_body(s_hbm, d_hbm, ew_hbm, x_hbm, w1_hbm, b1_hbm,
                up_hbm, degp_hbm, c0p_hbm, selS_hbm, selT_hbm, selW_hbm,
                s_ch, d_ch, ew_ch,
                nodef_a, nodef_b,
                degsl, c0sl, invsl, w0cmp, z4,
                stg_s, stg_t, stg_w,
                w1_loc, b1_loc,
                gidx4, sidx, xrowA, arow, u_loc,
                dinv_sh, inv_sh, agg_sh,
                sem, sem2):
    tid = lax.axis_index("s")
    i16 = lax.iota(jnp.int32, 16)
    zf = jnp.zeros((16,), jnp.float32)
    nb = tid * _NPT

    def z_big(i, c):
        nodef_a[pl.ds(i * 16, 16)] = zf
        nodef_b[pl.ds(i * 16, 16)] = zf
        return c
    lax.fori_loop(0, _NPAD // 16, z_big, 0)

    def z_small(i, c):
        degsl[pl.ds(i * 16, 16)] = zf
        c0sl[pl.ds(i * 16, 16)] = zf
        return c
    lax.fori_loop(0, _NPT // 16, z_small, 0)

    def z_u(i, c):
        u_loc[pl.ds(i * 16, 16)] = zf
        return c
    lax.fori_loop(0, (_T * _H) // 16, z_u, 0)
    for r in range(4):
        for q in range(_D // 16):
            z4[r, pl.ds(q * 16, 16)] = zf

    pltpu.sync_copy(w1_hbm, w1_loc)
    pltpu.sync_copy(b1_hbm, b1_loc)

    def p1_chunk(c, carry):
        base = pl.multiple_of(tid * _EPT + c * _CH, 8)
        pltpu.sync_copy(s_hbm.at[pl.ds(base, _CH)], s_ch)
        pltpu.sync_copy(d_hbm.at[pl.ds(base, _CH)], d_ch)
        pltpu.sync_copy(ew_hbm.at[pl.ds(base, _CH)], ew_ch)

        def p1_v(i, cc):
            sv = s_ch[pl.ds(i * 16, 16)]
            dv = d_ch[pl.ds(i * 16, 16)]
            ev = ew_ch[pl.ds(i * 16, 16)]
            plsc.addupdate_scatter(nodef_a, [dv], ev)
            plsc.addupdate_scatter(nodef_b, [sv],
                                   jnp.where(dv == 0, ev, jnp.float32(0.0)))
            return cc
        lax.fori_loop(0, _CH // 16, p1_v, 0)
        return carry
    lax.fori_loop(0, _NCH, p1_chunk, 0)

    descs = []
    for i in range(_NT):
        dsto = pl.multiple_of(i * _NPAD + tid * _NPT, 8)
        descs.append(pltpu.async_copy(
            nodef_a.at[pl.ds(i * _NPT, _NPT)],
            degp_hbm.at[pl.ds(dsto, _NPT)], sem))
        descs.append(pltpu.async_copy(
            nodef_b.at[pl.ds(i * _NPT, _NPT)],
            c0p_hbm.at[pl.ds(dsto, _NPT)], sem))
    for dsc in descs:
        dsc.wait()
    plsc.subcore_barrier()
    pltpu.sync_copy(degp_hbm.at[pl.ds(pl.multiple_of(tid * _NPAD, 8), _NPAD)],
                    nodef_a)
    pltpu.sync_copy(c0p_hbm.at[pl.ds(pl.multiple_of(tid * _NPAD, 8), _NPAD)],
                    nodef_b)
    for j in range(_NT):
        def accd(i, c):
            degsl[pl.ds(i * 16, 16)] = (degsl[pl.ds(i * 16, 16)]
                                        + nodef_a[pl.ds(j * _NPT + i * 16, 16)])
            c0sl[pl.ds(i * 16, 16)] = (c0sl[pl.ds(i * 16, 16)]
                                       + nodef_b[pl.ds(j * _NPT + i * 16, 16)])
            return c
        lax.fori_loop(0, _NPT // 16, accd, 0)

    def p3(i, c):
        dg = degsl[pl.ds(i * 16, 16)] + 1.0
        ib = plsc.bitcast(dg, jnp.int32)
        ib = 0x5F3759DF - (ib >> 1)
        y = plsc.bitcast(ib, jnp.float32)
        y = y * (1.5 - 0.5 * dg * y * y)
        y = y * (1.5 - 0.5 * dg * y * y)
        y = y * (1.5 - 0.5 * dg * y * y)
        degsl[pl.ds(i * 16, 16)] = y
        return c
    lax.fori_loop(0, _NPT // 16, p3, 0)
    pltpu.sync_copy(degsl, dinv_sh.at[pl.ds(pl.multiple_of(nb, 8), _NPT)])
    plsc.subcore_barrier()
    pltpu.sync_copy(dinv_sh, nodef_a)

    def _flush(off, cnt):
        for p in range(4):
            pos = cnt + p * 16 + i16
            plsc.store_scatter(stg_s, [pos], jnp.zeros((16,), jnp.int32))
            plsc.store_scatter(stg_t, [pos], jnp.full((16,), _DUMP, jnp.int32))
            plsc.store_scatter(stg_w, [pos], zf)
        padded = ((cnt + 63) >> 6) << 6

        def fl(p, c):
            dst = pl.multiple_of(tid * _ROW + off + p * 64, 8)
            pltpu.sync_copy(stg_s.at[pl.ds(p * 64, 64)],
                            selS_hbm.at[pl.ds(dst, 64)])
            pltpu.sync_copy(stg_t.at[pl.ds(p * 64, 64)],
                            selT_hbm.at[pl.ds(dst, 64)])
            pltpu.sync_copy(stg_w.at[pl.ds(p * 64, 64)],
                            selW_hbm.at[pl.ds(dst, 64)])
            return c
        lax.fori_loop(0, padded >> 6, fl, 0)
        return off + padded

    dinv0 = nodef_a[pl.ds(0, 16)][0]

    def p4(i, st):
        lslot = st
        gid = nb + i * 16 + i16
        c0v = c0sl[pl.ds(i * 16, 16)]
        dvv = degsl[pl.ds(i * 16, 16)]
        m = (c0v > 0.0) | (gid == 0)
        mi = m.astype(jnp.int32)
        excl = plsc.cumsum(mi) - mi
        cnt = jnp.sum(mi)
        slot_local = lslot + excl
        w0v = dinv0 * (c0v * dvv + jnp.where(gid == 0, dinv0, jnp.float32(0.0)))
        plsc.store_scatter(w0cmp, [slot_local], w0v, mask=m)
        invsl[pl.ds(i * 16, 16)] = jnp.where(
            m, (nb + slot_local + 1).astype(jnp.float32), jnp.float32(0.0))
        plsc.store_scatter(stg_s, [slot_local], gid, mask=m)
        plsc.store_scatter(stg_t, [slot_local], nb + slot_local, mask=m)
        plsc.store_scatter(stg_w, [slot_local], dvv * dvv, mask=m)
        return lslot + cnt
    cnt_own = lax.fori_loop(0, _NPT // 16, p4, jnp.int32(0))
    pltpu.sync_copy(invsl, inv_sh.at[pl.ds(pl.multiple_of(nb, 8), _NPT)])
    plsc.subcore_barrier()
    pltpu.sync_copy(inv_sh, nodef_b)

    def p5_chunk(c, carry):
        off, lc0 = carry
        base = pl.multiple_of(tid * _EPT + c * _CH, 8)
        pltpu.sync_copy(s_hbm.at[pl.ds(base, _CH)], s_ch)
        pltpu.sync_copy(d_hbm.at[pl.ds(base, _CH)], d_ch)
        pltpu.sync_copy(ew_hbm.at[pl.ds(base, _CH)], ew_ch)

        def p5_v(i, lc):
            sv = s_ch[pl.ds(i * 16, 16)]
            dv = d_ch[pl.ds(i * 16, 16)]
            ev = ew_ch[pl.ds(i * 16, 16)]
            g = plsc.load_gather(nodef_b, [dv])
            m = g > 0.5
            mi = m.astype(jnp.int32)
            app = lc + plsc.cumsum(mi) - mi
            nrm = (plsc.load_gather(nodef_a, [sv]) * ev *
                   plsc.load_gather(nodef_a, [dv]))
            plsc.store_scatter(stg_s, [app], sv, mask=m)
            plsc.store_scatter(stg_t, [app], (g - 1.0).astype(jnp.int32),
                               mask=m)
            plsc.store_scatter(stg_w, [app], nrm, mask=m)
            return lc + jnp.sum(mi)
        lc = lax.fori_loop(0, _CH // 16, p5_v, lc0)

        def spill(args):
            o, l = args
            for p in range(16):
                dst = pl.multiple_of(tid * _ROW + o + p * 64, 8)
                pltpu.sync_copy(stg_s.at[pl.ds(p * 64, 64)],
                                selS_hbm.at[pl.ds(dst, 64)])
                pltpu.sync_copy(stg_t.at[pl.ds(p * 64, 64)],
                                selT_hbm.at[pl.ds(dst, 64)])
                pltpu.sync_copy(stg_w.at[pl.ds(p * 64, 64)],
                                selW_hbm.at[pl.ds(dst, 64)])

            def sh(i, cc):
                stg_s[pl.ds(i * 16, 16)] = stg_s[pl.ds(1024 + i * 16, 16)]
                stg_t[pl.ds(i * 16, 16)] = stg_t[pl.ds(1024 + i * 16, 16)]
                stg_w[pl.ds(i * 16, 16)] = stg_w[pl.ds(1024 + i * 16, 16)]
                return cc
            lax.fori_loop(0, (l - 1024 + 15) >> 4, sh, 0)
            return (o + 1024, l - 1024)
        return lax.cond(lc >= 1024, spill, lambda a: a, (off, lc))
    off, lc = lax.fori_loop(0, _NCH, p5_chunk, (jnp.int32(0), cnt_own))
    off = _flush(off, lc)
    ngroups = off >> 4

    nch = (off + 1023) >> 10

    def load_sel_chunk(ci):
        cbase = pl.multiple_of(tid * _ROW + ci * 1024, 8)
        pltpu.sync_copy(selS_hbm.at[pl.ds(cbase, 1024)],
                        stg_s.at[pl.ds(0, 1024)])
        pltpu.sync_copy(selT_hbm.at[pl.ds(cbase, 1024)],
                        stg_t.at[pl.ds(0, 1024)])
        pltpu.sync_copy(selW_hbm.at[pl.ds(cbase, 1024)],
                        stg_w.at[pl.ds(0, 1024)])
    load_sel_chunk(0)

    def t_loop(t, carry):
        def za(jb, c):
            pltpu.sync_copy(z4, agg_sh.at[pl.ds(nb + jb * 4, 4)])
            return c
        lax.fori_loop(0, (cnt_own + 3) >> 2, za, 0)
        plsc.subcore_barrier()

        def cchunk(ci, c):
            @pl.when(jnp.logical_or(ci > 0, nch > 1))
            def _():
                load_sel_chunk(ci)
            ng_here = jnp.minimum(64, ngroups - ci * 64)

            def blk(bi, cc):
                g0 = bi * 2
                for b in range(2):
                    gidx4[pl.ds(b * 16, 16)] = (
                        stg_s[pl.ds((g0 + b) * 16, 16)] + t * _N)
                dsc0 = pltpu.async_copy(
                    x_hbm.at[gidx4.at[pl.ds(0, 16)]],
                    xrowA.at[pl.ds(0, 16)], sem2)
                dsc1 = pltpu.async_copy(
                    x_hbm.at[gidx4.at[pl.ds(16, 16)]],
                    xrowA.at[pl.ds(16, 16)], sem2)
                dsc0.wait()
                dsc1.wait()
                for b in range(2):
                    nv = stg_w[pl.ds((g0 + b) * 16, 16)]
                    for r in range(16):
                        nrm = nv[r]
                        for q in range(_D // 16):
                            xrowA[b * 16 + r, pl.ds(q * 16, 16)] = (
                                xrowA[b * 16 + r, pl.ds(q * 16, 16)] * nrm)
                    sidx[...] = stg_t[pl.ds((g0 + b) * 16, 16)]
                    pltpu.sync_copy(xrowA.at[pl.ds(b * 16, 16)],
                                    agg_sh.at[sidx], add=True)
                return cc
            lax.fori_loop(0, ng_here >> 1, blk, 0)
            return c
        lax.fori_loop(0, nch, cchunk, 0)
        plsc.subcore_barrier()

        def cs(j, c):
            pltpu.sync_copy(agg_sh.at[nb + j], arow)
            w0j = w0cmp[pl.ds(j, 16)][0]

            def kk(kb, acc):
                a0, a1, a2, a3 = acc
                av = arow[pl.ds(kb * 16, 16)]
                for r in range(16):
                    ak = av[r]
                    k = kb * 16 + r
                    a0 = a0 + ak * w1_loc[pl.ds(k * 64, 16)]
                    a1 = a1 + ak * w1_loc[pl.ds(k * 64 + 16, 16)]
                    a2 = a2 + ak * w1_loc[pl.ds(k * 64 + 32, 16)]
                    a3 = a3 + ak * w1_loc[pl.ds(k * 64 + 48, 16)]
                return (a0, a1, a2, a3)
            accs = lax.fori_loop(0, _D // 16, kk, (zf, zf, zf, zf))
            for q in range(4):
                hq = jnp.maximum(accs[q] + b1_loc[pl.ds(q * 16, 16)], 0.0)
                u_loc[pl.ds(t * 64 + q * 16, 16)] = (
                    u_loc[pl.ds(t * 64 + q * 16, 16)] + w0j * hq)
            return c
        lax.fori_loop(0, cnt_own, cs, 0)
        return carry
    lax.fori_loop(0, _T, t_loop, 0)

    pltpu.sync_copy(u_loc, up_hbm.at[pl.ds(pl.multiple_of(tid * _T * _H, 8), _T * _H)])


def _graph_call(s, d, ew, x2d, w1f, b1):
    return pl.kernel(
        _graph_body,
        out_type=(
            jax.ShapeDtypeStruct((_NT * _T * _H,), jnp.float32),
            jax.ShapeDtypeStruct((_NT * _NPAD,), jnp.float32),
            jax.ShapeDtypeStruct((_NT * _NPAD,), jnp.float32),
            jax.ShapeDtypeStruct((_NT * _ROW,), jnp.int32),
            jax.ShapeDtypeStruct((_NT * _ROW,), jnp.int32),
            jax.ShapeDtypeStruct((_NT * _ROW,), jnp.float32),
        ),
        mesh=plsc.VectorSubcoreMesh(core_axis_name="c", subcore_axis_name="s",
                                    num_cores=1),
        compiler_params=pltpu.CompilerParams(needs_layout_passes=False),
        scratch_types=[
            pltpu.VMEM((_CH,), jnp.int32),
            pltpu.VMEM((_CH,), jnp.int32),
            pltpu.VMEM((_CH,), jnp.float32),
            pltpu.VMEM((_NPAD,), jnp.float32),
            pltpu.VMEM((_NPAD,), jnp.float32),
            pltpu.VMEM((_NPT,), jnp.float32),
            pltpu.VMEM((_NPT,), jnp.float32),
            pltpu.VMEM((_NPT,), jnp.float32),
            pltpu.VMEM((_NPT + 16,), jnp.float32),
            pltpu.VMEM((4, _D), jnp.float32),
            pltpu.VMEM((_STG,), jnp.int32),
            pltpu.VMEM((_STG,), jnp.int32),
            pltpu.VMEM((_STG,), jnp.float32),
            pltpu.VMEM((_D * _H,), jnp.float32),
            pltpu.VMEM((_H,), jnp.float32),
            pltpu.VMEM((32,), jnp.int32),
            pltpu.VMEM((16,), jnp.int32),
            pltpu.VMEM((32, _D), jnp.float32),
            pltpu.VMEM((_D,), jnp.float32),
            pltpu.VMEM((_T * _H,), jnp.float32),
            pltpu.VMEM_SHARED((_NPAD,), jnp.float32),
            pltpu.VMEM_SHARED((_NPAD,), jnp.float32),
            pltpu.VMEM_SHARED((_AGG_ROWS, _D), jnp.float32),
            pltpu.SemaphoreType.DMA,
            pltpu.SemaphoreType.DMA,
        ],
    )(s, d, ew, x2d, w1f, b1)


def _head_body(m_ref, up_ref, w2_ref, b2_ref, wih_ref, whh_ref,
               bih_ref, bhh_ref, hwt_ref, hb_ref, out_ref):
    u = jnp.dot(m_ref[...], up_ref[...], preferred_element_type=jnp.float32)
    seq = jnp.maximum(
        jnp.dot(u, w2_ref[...], preferred_element_type=jnp.float32)
        + b2_ref[...][None, :], 0.0)
    h = jnp.zeros((1, _H), jnp.float32)
    for t in range(_T):
        xt = seq[t:t + 1, :]
        gi = jnp.dot(xt, wih_ref[...],
                     preferred_element_type=jnp.float32) + bih_ref[...][None, :]
        gh = jnp.dot(h, whh_ref[...],
                     preferred_element_type=jnp.float32) + bhh_ref[...][None, :]
        r = jax.nn.sigmoid(gi[:, :_H] + gh[:, :_H])
        z = jax.nn.sigmoid(gi[:, _H:2 * _H] + gh[:, _H:2 * _H])
        n = jnp.tanh(gi[:, 2 * _H:] + r * gh[:, 2 * _H:])
        h = (1.0 - z) * n + z * h
    logits = jnp.dot(h, hwt_ref[...],
                     preferred_element_type=jnp.float32) + hb_ref[...][None, :]
    out_ref[...] = jax.nn.sigmoid(logits)


def _head_call(m, up, W2, b2, wihT, whhT, b_ih, b_hh, hWT, heads_b):
    return pl.pallas_call(
        _head_body,
        out_shape=jax.ShapeDtypeStruct((1, 8), jnp.float32),
    )(m, up, W2, b2, wihT, whhT, b_ih, b_hh, hWT, heads_b)


def kernel(x, edge_index, edge_weight, W1, b1, W2, b2, W_ih, W_hh,
           b_ih, b_hh, heads_W, heads_b):
    s = edge_index[0].astype(jnp.int32)
    d = edge_index[1].astype(jnp.int32)
    x2d = x.reshape(_T * _N, _D)
    outs = _graph_call(s, d, edge_weight.astype(jnp.float32), x2d,
                       W1.reshape(-1), b1)
    up = outs[0].reshape(_NT * _T, _H)
    m = jnp.tile(jnp.eye(_T, dtype=jnp.float32), (1, _NT))
    return _head_call(m, up, W2, b2, W_ih.T, W_hh.T, b_ih, b_hh,
                      heads_W.T, heads_b)


# speedup vs baseline: 279.1905x; 1.1405x over previous
import jax
import jax.numpy as jnp
from jax import lax
from jax.experimental import pallas as pl
from jax.experimental.pallas import tpu as pltpu
from jax.experimental.pallas import tpu_sc as plsc

_T, _N, _D, _H = 12, 10000, 128, 64
_E = 320000
_NT = 16
_NPT = 640
_NPAD = _NT * _NPT
_EPT = _E // _NT
_CH = 1000
_NCH = _EPT // _CH
_STG = 2112
_ROW = 22528
_DUMP = _NPAD
_AGG_ROWS = _NPAD + 8


def _graph_body(s_hbm, d_hbm, ew_hbm, x_hbm, w1_hbm, b1_hbm,
                up_hbm, degp_hbm, c0p_hbm, selS_hbm, selT_hbm, selW_hbm,
                s_ch, d_ch, ew_ch,
                nodef_a, nodef_b,
                degsl, c0sl, invsl, w0cmp, z4,
                stg_s, stg_t, stg_w,
                w1_loc, b1_loc,
                gidx4, sidx, xrowA, arow, arow2, u_loc,
                dinv_sh, inv_sh, agg_sh,
                sem, sem2):
    tid = lax.axis_index("s")
    i16 = lax.iota(jnp.int32, 16)
    zf = jnp.zeros((16,), jnp.float32)
    nb = tid * _NPT

    def z_big(i, c):
        nodef_a[pl.ds(i * 16, 16)] = zf
        nodef_b[pl.ds(i * 16, 16)] = zf
        return c
    lax.fori_loop(0, _NPAD // 16, z_big, 0)

    def z_small(i, c):
        degsl[pl.ds(i * 16, 16)] = zf
        c0sl[pl.ds(i * 16, 16)] = zf
        return c
    lax.fori_loop(0, _NPT // 16, z_small, 0)

    def z_u(i, c):
        u_loc[pl.ds(i * 16, 16)] = zf
        return c
    lax.fori_loop(0, (_T * _H) // 16, z_u, 0)
    for r in range(4):
        for q in range(_D // 16):
            z4[r, pl.ds(q * 16, 16)] = zf

    pltpu.sync_copy(w1_hbm, w1_loc)
    pltpu.sync_copy(b1_hbm, b1_loc)

    def p1_chunk(c, carry):
        base = pl.multiple_of(tid * _EPT + c * _CH, 8)
        d0 = pltpu.async_copy(s_hbm.at[pl.ds(base, _CH)], s_ch, sem)
        d1 = pltpu.async_copy(d_hbm.at[pl.ds(base, _CH)], d_ch, sem)
        d2 = pltpu.async_copy(ew_hbm.at[pl.ds(base, _CH)], ew_ch, sem)
        d0.wait(); d1.wait(); d2.wait()

        def p1_v(i, cc):
            sv = s_ch[pl.ds(i * 16, 16)]
            dv = d_ch[pl.ds(i * 16, 16)]
            ev = ew_ch[pl.ds(i * 16, 16)]
            plsc.addupdate_scatter(nodef_a, [dv], ev)
            plsc.addupdate_scatter(nodef_b, [sv],
                                   jnp.where(dv == 0, ev, jnp.float32(0.0)))
            return cc
        lax.fori_loop(0, _CH // 16, p1_v, 0)
        return carry
    lax.fori_loop(0, _NCH, p1_chunk, 0)

    descs = []
    for i in range(_NT):
        dsto = pl.multiple_of(i * _NPAD + tid * _NPT, 8)
        descs.append(pltpu.async_copy(
            nodef_a.at[pl.ds(i * _NPT, _NPT)],
            degp_hbm.at[pl.ds(dsto, _NPT)], sem))
        descs.append(pltpu.async_copy(
            nodef_b.at[pl.ds(i * _NPT, _NPT)],
            c0p_hbm.at[pl.ds(dsto, _NPT)], sem))
    for dsc in descs:
        dsc.wait()
    plsc.subcore_barrier()
    pltpu.sync_copy(degp_hbm.at[pl.ds(pl.multiple_of(tid * _NPAD, 8), _NPAD)],
                    nodef_a)
    pltpu.sync_copy(c0p_hbm.at[pl.ds(pl.multiple_of(tid * _NPAD, 8), _NPAD)],
                    nodef_b)
    for j in range(_NT):
        def accd(i, c):
            degsl[pl.ds(i * 16, 16)] = (degsl[pl.ds(i * 16, 16)]
                                        + nodef_a[pl.ds(j * _NPT + i * 16, 16)])
            c0sl[pl.ds(i * 16, 16)] = (c0sl[pl.ds(i * 16, 16)]
                                       + nodef_b[pl.ds(j * _NPT + i * 16, 16)])
            return c
        lax.fori_loop(0, _NPT // 16, accd, 0)

    def p3(i, c):
        dg = degsl[pl.ds(i * 16, 16)] + 1.0
        ib = plsc.bitcast(dg, jnp.int32)
        ib = 0x5F3759DF - (ib >> 1)
        y = plsc.bitcast(ib, jnp.float32)
        y = y * (1.5 - 0.5 * dg * y * y)
        y = y * (1.5 - 0.5 * dg * y * y)
        y = y * (1.5 - 0.5 * dg * y * y)
        degsl[pl.ds(i * 16, 16)] = y
        return c
    lax.fori_loop(0, _NPT // 16, p3, 0)
    pltpu.sync_copy(degsl, dinv_sh.at[pl.ds(pl.multiple_of(nb, 8), _NPT)])
    plsc.subcore_barrier()
    pltpu.sync_copy(dinv_sh, nodef_a)

    def _flush(off, cnt):
        for p in range(4):
            pos = cnt + p * 16 + i16
            plsc.store_scatter(stg_s, [pos], jnp.zeros((16,), jnp.int32))
            plsc.store_scatter(stg_t, [pos], jnp.full((16,), _DUMP, jnp.int32))
            plsc.store_scatter(stg_w, [pos], zf)
        padded = ((cnt + 63) >> 6) << 6

        def fl(p, c):
            dst = pl.multiple_of(tid * _ROW + off + p * 64, 8)
            pltpu.sync_copy(stg_s.at[pl.ds(p * 64, 64)],
                            selS_hbm.at[pl.ds(dst, 64)])
            pltpu.sync_copy(stg_t.at[pl.ds(p * 64, 64)],
                            selT_hbm.at[pl.ds(dst, 64)])
            pltpu.sync_copy(stg_w.at[pl.ds(p * 64, 64)],
                            selW_hbm.at[pl.ds(dst, 64)])
            return c
        lax.fori_loop(0, padded >> 6, fl, 0)
        return off + padded

    dinv0 = nodef_a[pl.ds(0, 16)][0]

    def p4(i, st):
        lslot = st
        gid = nb + i * 16 + i16
        c0v = c0sl[pl.ds(i * 16, 16)]
        dvv = degsl[pl.ds(i * 16, 16)]
        m = (c0v > 0.0) | (gid == 0)
        mi = m.astype(jnp.int32)
        excl = plsc.cumsum(mi) - mi
        cnt = jnp.sum(mi)
        slot_local = lslot + excl
        w0v = dinv0 * (c0v * dvv + jnp.where(gid == 0, dinv0, jnp.float32(0.0)))
        plsc.store_scatter(w0cmp, [slot_local], w0v, mask=m)
        invsl[pl.ds(i * 16, 16)] = jnp.where(
            m, (nb + slot_local + 1).astype(jnp.float32), jnp.float32(0.0))
        plsc.store_scatter(stg_s, [slot_local], gid, mask=m)
        plsc.store_scatter(stg_t, [slot_local], nb + slot_local, mask=m)
        plsc.store_scatter(stg_w, [slot_local], dvv * dvv, mask=m)
        return lslot + cnt
    cnt_own = lax.fori_loop(0, _NPT // 16, p4, jnp.int32(0))
    pltpu.sync_copy(invsl, inv_sh.at[pl.ds(pl.multiple_of(nb, 8), _NPT)])
    plsc.subcore_barrier()
    pltpu.sync_copy(inv_sh, nodef_b)

    def p5_chunk(c, carry):
        off, lc0 = carry
        base = pl.multiple_of(tid * _EPT + c * _CH, 8)
        d0 = pltpu.async_copy(s_hbm.at[pl.ds(base, _CH)], s_ch, sem)
        d1 = pltpu.async_copy(d_hbm.at[pl.ds(base, _CH)], d_ch, sem)
        d2 = pltpu.async_copy(ew_hbm.at[pl.ds(base, _CH)], ew_ch, sem)
        d0.wait(); d1.wait(); d2.wait()

        def p5_v(i, lc):
            sv = s_ch[pl.ds(i * 16, 16)]
            dv = d_ch[pl.ds(i * 16, 16)]
            ev = ew_ch[pl.ds(i * 16, 16)]
            g = plsc.load_gather(nodef_b, [dv])
            m = g > 0.5
            mi = m.astype(jnp.int32)
            app = lc + plsc.cumsum(mi) - mi
            nrm = (plsc.load_gather(nodef_a, [sv]) * ev *
                   plsc.load_gather(nodef_a, [dv]))
            plsc.store_scatter(stg_s, [app], sv, mask=m)
            plsc.store_scatter(stg_t, [app], (g - 1.0).astype(jnp.int32),
                               mask=m)
            plsc.store_scatter(stg_w, [app], nrm, mask=m)
            return lc + jnp.sum(mi)
        lc = lax.fori_loop(0, _CH // 16, p5_v, lc0)

        def spill(args):
            o, l = args
            for p in range(16):
                dst = pl.multiple_of(tid * _ROW + o + p * 64, 8)
                pltpu.sync_copy(stg_s.at[pl.ds(p * 64, 64)],
                                selS_hbm.at[pl.ds(dst, 64)])
                pltpu.sync_copy(stg_t.at[pl.ds(p * 64, 64)],
                                selT_hbm.at[pl.ds(dst, 64)])
                pltpu.sync_copy(stg_w.at[pl.ds(p * 64, 64)],
                                selW_hbm.at[pl.ds(dst, 64)])

            def sh(i, cc):
                stg_s[pl.ds(i * 16, 16)] = stg_s[pl.ds(1024 + i * 16, 16)]
                stg_t[pl.ds(i * 16, 16)] = stg_t[pl.ds(1024 + i * 16, 16)]
                stg_w[pl.ds(i * 16, 16)] = stg_w[pl.ds(1024 + i * 16, 16)]
                return cc
            lax.fori_loop(0, (l - 1024 + 15) >> 4, sh, 0)
            return (o + 1024, l - 1024)
        return lax.cond(lc >= 1024, spill, lambda a: a, (off, lc))
    off, lc = lax.fori_loop(0, _NCH, p5_chunk, (jnp.int32(0), cnt_own))
    off = _flush(off, lc)
    ngroups = off >> 4

    nch = (off + 1023) >> 10

    def load_sel_chunk(ci):
        cbase = pl.multiple_of(tid * _ROW + ci * 1024, 8)
        d0 = pltpu.async_copy(selS_hbm.at[pl.ds(cbase, 1024)],
                              stg_s.at[pl.ds(0, 1024)], sem)
        d1 = pltpu.async_copy(selT_hbm.at[pl.ds(cbase, 1024)],
                              stg_t.at[pl.ds(0, 1024)], sem)
        d2 = pltpu.async_copy(selW_hbm.at[pl.ds(cbase, 1024)],
                              stg_w.at[pl.ds(0, 1024)], sem)
        d0.wait(); d1.wait(); d2.wait()
    load_sel_chunk(0)

    def t_loop(t, carry):
        def za(jb, c):
            pltpu.sync_copy(z4, agg_sh.at[pl.ds(nb + jb * 4, 4)])
            return c
        lax.fori_loop(0, (cnt_own + 3) >> 2, za, 0)
        plsc.subcore_barrier()

        def cchunk(ci, c):
            @pl.when(jnp.logical_or(ci > 0, nch > 1))
            def _():
                load_sel_chunk(ci)
            ng_here = jnp.minimum(64, ngroups - ci * 64)

            def blk(bi, cc):
                g0 = bi * 2
                for b in range(2):
                    gidx4[pl.ds(b * 16, 16)] = (
                        stg_s[pl.ds((g0 + b) * 16, 16)] + t * _N)
                dsc0 = pltpu.async_copy(
                    x_hbm.at[gidx4.at[pl.ds(0, 16)]],
                    xrowA.at[pl.ds(0, 16)], sem2)
                dsc1 = pltpu.async_copy(
                    x_hbm.at[gidx4.at[pl.ds(16, 16)]],
                    xrowA.at[pl.ds(16, 16)], sem2)
                dsc0.wait()
                dsc1.wait()
                for b in range(2):
                    nv = stg_w[pl.ds((g0 + b) * 16, 16)]
                    for r in range(16):
                        nrm = nv[r]
                        for q in range(_D // 16):
                            xrowA[b * 16 + r, pl.ds(q * 16, 16)] = (
                                xrowA[b * 16 + r, pl.ds(q * 16, 16)] * nrm)
                    sidx[...] = stg_t[pl.ds((g0 + b) * 16, 16)]
                    pltpu.sync_copy(xrowA.at[pl.ds(b * 16, 16)],
                                    agg_sh.at[sidx], add=True)
                return cc
            lax.fori_loop(0, ng_here >> 1, blk, 0)
            return c
        lax.fori_loop(0, nch, cchunk, 0)
        plsc.subcore_barrier()

        def consume_row(buf, j):
            w0j = w0cmp[pl.ds(j, 16)][0]

            def kk(kb, acc):
                a0, a1, a2, a3 = acc
                av = buf[pl.ds(kb * 16, 16)]
                for r in range(16):
                    ak = av[r]
                    k = kb * 16 + r
                    a0 = a0 + ak * w1_loc[pl.ds(k * 64, 16)]
                    a1 = a1 + ak * w1_loc[pl.ds(k * 64 + 16, 16)]
                    a2 = a2 + ak * w1_loc[pl.ds(k * 64 + 32, 16)]
                    a3 = a3 + ak * w1_loc[pl.ds(k * 64 + 48, 16)]
                return (a0, a1, a2, a3)
            accs = lax.fori_loop(0, _D // 16, kk, (zf, zf, zf, zf))
            for q in range(4):
                hq = jnp.maximum(accs[q] + b1_loc[pl.ds(q * 16, 16)], 0.0)
                u_loc[pl.ds(t * 64 + q * 16, 16)] = (
                    u_loc[pl.ds(t * 64 + q * 16, 16)] + w0j * hq)

        def cs(jb, c):
            j0 = jb * 2
            have2 = (j0 + 1) < cnt_own
            da = pltpu.async_copy(agg_sh.at[nb + j0], arow, sem2)

            @pl.when(have2)
            def _():
                pltpu.async_copy(agg_sh.at[nb + j0 + 1], arow2, sem2)
            da.wait()
            consume_row(arow, j0)

            @pl.when(have2)
            def _():
                pltpu.make_async_copy(agg_sh.at[nb + j0 + 1], arow2,
                                      sem2).wait()
                consume_row(arow2, j0 + 1)
            return c
        lax.fori_loop(0, (cnt_own + 1) >> 1, cs, 0)
        return carry
    lax.fori_loop(0, _T, t_loop, 0)

    pltpu.sync_copy(u_loc, up_hbm.at[pl.ds(pl.multiple_of(tid * _T * _H, 8), _T * _H)])


def _graph_call(s, d, ew, x2d, w1f, b1):
    return pl.kernel(
        _graph_body,
        out_type=(
            jax.ShapeDtypeStruct((_NT * _T * _H,), jnp.float32),
            jax.ShapeDtypeStruct((_NT * _NPAD,), jnp.float32),
            jax.ShapeDtypeStruct((_NT * _NPAD,), jnp.float32),
            jax.ShapeDtypeStruct((_NT * _ROW,), jnp.int32),
            jax.ShapeDtypeStruct((_NT * _ROW,), jnp.int32),
            jax.ShapeDtypeStruct((_NT * _ROW,), jnp.float32),
        ),
        mesh=plsc.VectorSubcoreMesh(core_axis_name="c", subcore_axis_name="s",
                                    num_cores=1),
        compiler_params=pltpu.CompilerParams(needs_layout_passes=False),
        scratch_types=[
            pltpu.VMEM((_CH,), jnp.int32),
            pltpu.VMEM((_CH,), jnp.int32),
            pltpu.VMEM((_CH,), jnp.float32),
            pltpu.VMEM((_NPAD,), jnp.float32),
            pltpu.VMEM((_NPAD,), jnp.float32),
            pltpu.VMEM((_NPT,), jnp.float32),
            pltpu.VMEM((_NPT,), jnp.float32),
            pltpu.VMEM((_NPT,), jnp.float32),
            pltpu.VMEM((_NPT + 16,), jnp.float32),
            pltpu.VMEM((4, _D), jnp.float32),
            pltpu.VMEM((_STG,), jnp.int32),
            pltpu.VMEM((_STG,), jnp.int32),
            pltpu.VMEM((_STG,), jnp.float32),
            pltpu.VMEM((_D * _H,), jnp.float32),
            pltpu.VMEM((_H,), jnp.float32),
            pltpu.VMEM((32,), jnp.int32),
            pltpu.VMEM((16,), jnp.int32),
            pltpu.VMEM((32, _D), jnp.float32),
            pltpu.VMEM((_D,), jnp.float32),
            pltpu.VMEM((_D,), jnp.float32),
            pltpu.VMEM((_T * _H,), jnp.float32),
            pltpu.VMEM_SHARED((_NPAD,), jnp.float32),
            pltpu.VMEM_SHARED((_NPAD,), jnp.float32),
            pltpu.VMEM_SHARED((_AGG_ROWS, _D), jnp.float32),
            pltpu.SemaphoreType.DMA,
            pltpu.SemaphoreType.DMA,
        ],
    )(s, d, ew, x2d, w1f, b1)


def _head_body(m_ref, up_ref, w2_ref, b2_ref, wih_ref, whh_ref,
               bih_ref, bhh_ref, hwt_ref, hb_ref, out_ref):
    u = jnp.dot(m_ref[...], up_ref[...], preferred_element_type=jnp.float32)
    seq = jnp.maximum(
        jnp.dot(u, w2_ref[...], preferred_element_type=jnp.float32)
        + b2_ref[...][None, :], 0.0)
    h = jnp.zeros((1, _H), jnp.float32)
    for t in range(_T):
        xt = seq[t:t + 1, :]
        gi = jnp.dot(xt, wih_ref[...],
                     preferred_element_type=jnp.float32) + bih_ref[...][None, :]
        gh = jnp.dot(h, whh_ref[...],
                     preferred_element_type=jnp.float32) + bhh_ref[...][None, :]
        r = jax.nn.sigmoid(gi[:, :_H] + gh[:, :_H])
        z = jax.nn.sigmoid(gi[:, _H:2 * _H] + gh[:, _H:2 * _H])
        n = jnp.tanh(gi[:, 2 * _H:] + r * gh[:, 2 * _H:])
        h = (1.0 - z) * n + z * h
    logits = jnp.dot(h, hwt_ref[...],
                     preferred_element_type=jnp.float32) + hb_ref[...][None, :]
    out_ref[...] = jax.nn.sigmoid(logits)


def _head_call(m, up, W2, b2, wihT, whhT, b_ih, b_hh, hWT, heads_b):
    return pl.pallas_call(
        _head_body,
        out_shape=jax.ShapeDtypeStruct((1, 8), jnp.float32),
    )(m, up, W2, b2, wihT, whhT, b_ih, b_hh, hWT, heads_b)


def kernel(x, edge_index, edge_weight, W1, b1, W2, b2, W_ih, W_hh,
           b_ih, b_hh, heads_W, heads_b):
    s = edge_index[0].astype(jnp.int32)
    d = edge_index[1].astype(jnp.int32)
    x2d = x.reshape(_T * _N, _D)
    outs = _graph_call(s, d, edge_weight.astype(jnp.float32), x2d,
                       W1.reshape(-1), b1)
    up = outs[0].reshape(_NT * _T, _H)
    m = jnp.tile(jnp.eye(_T, dtype=jnp.float32), (1, _NT))
    return _head_call(m, up, W2, b2, W_ih.T, W_hh.T, b_ih, b_hh,
                      heads_W.T, heads_b)


# speedup vs baseline: 280.7141x; 1.0055x over previous
import jax
import jax.numpy as jnp
from jax import lax
from jax.experimental import pallas as pl
from jax.experimental.pallas import tpu as pltpu
from jax.experimental.pallas import tpu_sc as plsc

_T, _N, _D, _H = 12, 10000, 128, 64
_E = 320000
_NT = 16
_NPT = 640
_NPAD = _NT * _NPT
_EPT = _E // _NT
_CH = 1000
_NCH = _EPT // _CH
_STG = 2112
_ROW = 22528
_DUMP = _NPAD
_AGG_ROWS = _NPAD + 8


def _graph_body(s_hbm, d_hbm, ew_hbm, x_hbm, w1_hbm, b1_hbm,
                up_hbm, degp_hbm, c0p_hbm, selS_hbm, selT_hbm, selW_hbm,
                s_ch, d_ch, ew_ch,
                nodef_a, nodef_b,
                degsl, c0sl, invsl, w0cmp, z4,
                stg_s, stg_t, stg_w,
                w1_loc, b1_loc,
                gidx4, sidx, xrowA, arow, arow2, u_loc,
                dinv_sh, inv_sh, agg_sh,
                sem, sem2):
    tid = lax.axis_index("s")
    i16 = lax.iota(jnp.int32, 16)
    zf = jnp.zeros((16,), jnp.float32)
    nb = tid * _NPT

    def z_big(i, c):
        nodef_a[pl.ds(i * 16, 16)] = zf
        nodef_b[pl.ds(i * 16, 16)] = zf
        return c
    lax.fori_loop(0, _NPAD // 16, z_big, 0, unroll=4)

    def z_small(i, c):
        degsl[pl.ds(i * 16, 16)] = zf
        c0sl[pl.ds(i * 16, 16)] = zf
        return c
    lax.fori_loop(0, _NPT // 16, z_small, 0)

    def z_u(i, c):
        u_loc[pl.ds(i * 16, 16)] = zf
        return c
    lax.fori_loop(0, (_T * _H) // 16, z_u, 0)
    for r in range(4):
        for q in range(_D // 16):
            z4[r, pl.ds(q * 16, 16)] = zf

    pltpu.sync_copy(w1_hbm, w1_loc)
    pltpu.sync_copy(b1_hbm, b1_loc)

    def p1_chunk(c, carry):
        base = pl.multiple_of(tid * _EPT + c * _CH, 8)
        d0 = pltpu.async_copy(s_hbm.at[pl.ds(base, _CH)], s_ch, sem)
        d1 = pltpu.async_copy(d_hbm.at[pl.ds(base, _CH)], d_ch, sem)
        d2 = pltpu.async_copy(ew_hbm.at[pl.ds(base, _CH)], ew_ch, sem)
        d0.wait(); d1.wait(); d2.wait()

        def p1_v(i, cc):
            sv = s_ch[pl.ds(i * 16, 16)]
            dv = d_ch[pl.ds(i * 16, 16)]
            ev = ew_ch[pl.ds(i * 16, 16)]
            plsc.addupdate_scatter(nodef_a, [dv], ev)
            plsc.addupdate_scatter(nodef_b, [sv],
                                   jnp.where(dv == 0, ev, jnp.float32(0.0)))
            return cc
        lax.fori_loop(0, _CH // 16, p1_v, 0, unroll=4)
        return carry
    lax.fori_loop(0, _NCH, p1_chunk, 0)

    descs = []
    for i in range(_NT):
        dsto = pl.multiple_of(i * _NPAD + tid * _NPT, 8)
        descs.append(pltpu.async_copy(
            nodef_a.at[pl.ds(i * _NPT, _NPT)],
            degp_hbm.at[pl.ds(dsto, _NPT)], sem))
        descs.append(pltpu.async_copy(
            nodef_b.at[pl.ds(i * _NPT, _NPT)],
            c0p_hbm.at[pl.ds(dsto, _NPT)], sem))
    for dsc in descs:
        dsc.wait()
    plsc.subcore_barrier()
    pltpu.sync_copy(degp_hbm.at[pl.ds(pl.multiple_of(tid * _NPAD, 8), _NPAD)],
                    nodef_a)
    pltpu.sync_copy(c0p_hbm.at[pl.ds(pl.multiple_of(tid * _NPAD, 8), _NPAD)],
                    nodef_b)
    for j in range(_NT):
        def accd(i, c):
            degsl[pl.ds(i * 16, 16)] = (degsl[pl.ds(i * 16, 16)]
                                        + nodef_a[pl.ds(j * _NPT + i * 16, 16)])
            c0sl[pl.ds(i * 16, 16)] = (c0sl[pl.ds(i * 16, 16)]
                                       + nodef_b[pl.ds(j * _NPT + i * 16, 16)])
            return c
        lax.fori_loop(0, _NPT // 16, accd, 0, unroll=4)

    def p3(i, c):
        dg = degsl[pl.ds(i * 16, 16)] + 1.0
        ib = plsc.bitcast(dg, jnp.int32)
        ib = 0x5F3759DF - (ib >> 1)
        y = plsc.bitcast(ib, jnp.float32)
        y = y * (1.5 - 0.5 * dg * y * y)
        y = y * (1.5 - 0.5 * dg * y * y)
        y = y * (1.5 - 0.5 * dg * y * y)
        degsl[pl.ds(i * 16, 16)] = y
        return c
    lax.fori_loop(0, _NPT // 16, p3, 0)
    pltpu.sync_copy(degsl, dinv_sh.at[pl.ds(pl.multiple_of(nb, 8), _NPT)])
    plsc.subcore_barrier()
    pltpu.sync_copy(dinv_sh, nodef_a)

    def _flush(off, cnt):
        for p in range(4):
            pos = cnt + p * 16 + i16
            plsc.store_scatter(stg_s, [pos], jnp.zeros((16,), jnp.int32))
            plsc.store_scatter(stg_t, [pos], jnp.full((16,), _DUMP, jnp.int32))
            plsc.store_scatter(stg_w, [pos], zf)
        padded = ((cnt + 63) >> 6) << 6

        def fl(p, c):
            dst = pl.multiple_of(tid * _ROW + off + p * 64, 8)
            pltpu.sync_copy(stg_s.at[pl.ds(p * 64, 64)],
                            selS_hbm.at[pl.ds(dst, 64)])
            pltpu.sync_copy(stg_t.at[pl.ds(p * 64, 64)],
                            selT_hbm.at[pl.ds(dst, 64)])
            pltpu.sync_copy(stg_w.at[pl.ds(p * 64, 64)],
                            selW_hbm.at[pl.ds(dst, 64)])
            return c
        lax.fori_loop(0, padded >> 6, fl, 0)
        return off + padded

    dinv0 = nodef_a[pl.ds(0, 16)][0]

    def p4(i, st):
        lslot = st
        gid = nb + i * 16 + i16
        c0v = c0sl[pl.ds(i * 16, 16)]
        dvv = degsl[pl.ds(i * 16, 16)]
        m = (c0v > 0.0) | (gid == 0)
        mi = m.astype(jnp.int32)
        excl = plsc.cumsum(mi) - mi
        cnt = jnp.sum(mi)
        slot_local = lslot + excl
        w0v = dinv0 * (c0v * dvv + jnp.where(gid == 0, dinv0, jnp.float32(0.0)))
        plsc.store_scatter(w0cmp, [slot_local], w0v, mask=m)
        invsl[pl.ds(i * 16, 16)] = jnp.where(
            m, (nb + slot_local + 1).astype(jnp.float32), jnp.float32(0.0))
        plsc.store_scatter(stg_s, [slot_local], gid, mask=m)
        plsc.store_scatter(stg_t, [slot_local], nb + slot_local, mask=m)
        plsc.store_scatter(stg_w, [slot_local], dvv * dvv, mask=m)
        return lslot + cnt
    cnt_own = lax.fori_loop(0, _NPT // 16, p4, jnp.int32(0))
    pltpu.sync_copy(invsl, inv_sh.at[pl.ds(pl.multiple_of(nb, 8), _NPT)])
    plsc.subcore_barrier()
    pltpu.sync_copy(inv_sh, nodef_b)

    def p5_chunk(c, carry):
        off, lc0 = carry
        base = pl.multiple_of(tid * _EPT + c * _CH, 8)
        d0 = pltpu.async_copy(s_hbm.at[pl.ds(base, _CH)], s_ch, sem)
        d1 = pltpu.async_copy(d_hbm.at[pl.ds(base, _CH)], d_ch, sem)
        d2 = pltpu.async_copy(ew_hbm.at[pl.ds(base, _CH)], ew_ch, sem)
        d0.wait(); d1.wait(); d2.wait()

        def p5_v(i, lc):
            sv = s_ch[pl.ds(i * 16, 16)]
            dv = d_ch[pl.ds(i * 16, 16)]
            ev = ew_ch[pl.ds(i * 16, 16)]
            g = plsc.load_gather(nodef_b, [dv])
            m = g > 0.5
            mi = m.astype(jnp.int32)
            app = lc + plsc.cumsum(mi) - mi
            nrm = (plsc.load_gather(nodef_a, [sv]) * ev *
                   plsc.load_gather(nodef_a, [dv]))
            plsc.store_scatter(stg_s, [app], sv, mask=m)
            plsc.store_scatter(stg_t, [app], (g - 1.0).astype(jnp.int32),
                               mask=m)
            plsc.store_scatter(stg_w, [app], nrm, mask=m)
            return lc + jnp.sum(mi)
        lc = lax.fori_loop(0, _CH // 16, p5_v, lc0, unroll=2)

        def spill(args):
            o, l = args
            for p in range(16):
                dst = pl.multiple_of(tid * _ROW + o + p * 64, 8)
                pltpu.sync_copy(stg_s.at[pl.ds(p * 64, 64)],
                                selS_hbm.at[pl.ds(dst, 64)])
                pltpu.sync_copy(stg_t.at[pl.ds(p * 64, 64)],
                                selT_hbm.at[pl.ds(dst, 64)])
                pltpu.sync_copy(stg_w.at[pl.ds(p * 64, 64)],
                                selW_hbm.at[pl.ds(dst, 64)])

            def sh(i, cc):
                stg_s[pl.ds(i * 16, 16)] = stg_s[pl.ds(1024 + i * 16, 16)]
                stg_t[pl.ds(i * 16, 16)] = stg_t[pl.ds(1024 + i * 16, 16)]
                stg_w[pl.ds(i * 16, 16)] = stg_w[pl.ds(1024 + i * 16, 16)]
                return cc
            lax.fori_loop(0, (l - 1024 + 15) >> 4, sh, 0)
            return (o + 1024, l - 1024)
        return lax.cond(lc >= 1024, spill, lambda a: a, (off, lc))
    off, lc = lax.fori_loop(0, _NCH, p5_chunk, (jnp.int32(0), cnt_own))
    off = _flush(off, lc)
    ngroups = off >> 4

    nch = (off + 1023) >> 10

    def load_sel_chunk(ci):
        cbase = pl.multiple_of(tid * _ROW + ci * 1024, 8)
        d0 = pltpu.async_copy(selS_hbm.at[pl.ds(cbase, 1024)],
                              stg_s.at[pl.ds(0, 1024)], sem)
        d1 = pltpu.async_copy(selT_hbm.at[pl.ds(cbase, 1024)],
                              stg_t.at[pl.ds(0, 1024)], sem)
        d2 = pltpu.async_copy(selW_hbm.at[pl.ds(cbase, 1024)],
                              stg_w.at[pl.ds(0, 1024)], sem)
        d0.wait(); d1.wait(); d2.wait()
    load_sel_chunk(0)

    def t_loop(t, carry):
        def za(jb, c):
            pltpu.sync_copy(z4, agg_sh.at[pl.ds(nb + jb * 4, 4)])
            return c
        lax.fori_loop(0, (cnt_own + 3) >> 2, za, 0)
        plsc.subcore_barrier()

        def cchunk(ci, c):
            @pl.when(jnp.logical_or(ci > 0, nch > 1))
            def _():
                load_sel_chunk(ci)
            ng_here = jnp.minimum(64, ngroups - ci * 64)

            def blk(bi, cc):
                g0 = bi * 2
                for b in range(2):
                    gidx4[pl.ds(b * 16, 16)] = (
                        stg_s[pl.ds((g0 + b) * 16, 16)] + t * _N)
                dsc0 = pltpu.async_copy(
                    x_hbm.at[gidx4.at[pl.ds(0, 16)]],
                    xrowA.at[pl.ds(0, 16)], sem2)
                dsc1 = pltpu.async_copy(
                    x_hbm.at[gidx4.at[pl.ds(16, 16)]],
                    xrowA.at[pl.ds(16, 16)], sem2)
                dsc0.wait()
                dsc1.wait()
                for b in range(2):
                    nv = stg_w[pl.ds((g0 + b) * 16, 16)]
                    for r in range(16):
                        nrm = nv[r]
                        for q in range(_D // 16):
                            xrowA[b * 16 + r, pl.ds(q * 16, 16)] = (
                                xrowA[b * 16 + r, pl.ds(q * 16, 16)] * nrm)
                    sidx[...] = stg_t[pl.ds((g0 + b) * 16, 16)]
                    pltpu.sync_copy(xrowA.at[pl.ds(b * 16, 16)],
                                    agg_sh.at[sidx], add=True)
                return cc
            lax.fori_loop(0, ng_here >> 1, blk, 0)
            return c
        lax.fori_loop(0, nch, cchunk, 0)
        plsc.subcore_barrier()

        def consume_row(buf, j):
            w0j = w0cmp[pl.ds(j, 16)][0]

            def kk(kb, acc):
                a0, a1, a2, a3 = acc
                av = buf[pl.ds(kb * 16, 16)]
                for r in range(16):
                    ak = av[r]
                    k = kb * 16 + r
                    a0 = a0 + ak * w1_loc[pl.ds(k * 64, 16)]
                    a1 = a1 + ak * w1_loc[pl.ds(k * 64 + 16, 16)]
                    a2 = a2 + ak * w1_loc[pl.ds(k * 64 + 32, 16)]
                    a3 = a3 + ak * w1_loc[pl.ds(k * 64 + 48, 16)]
                return (a0, a1, a2, a3)
            accs = lax.fori_loop(0, _D // 16, kk, (zf, zf, zf, zf))
            for q in range(4):
                hq = jnp.maximum(accs[q] + b1_loc[pl.ds(q * 16, 16)], 0.0)
                u_loc[pl.ds(t * 64 + q * 16, 16)] = (
                    u_loc[pl.ds(t * 64 + q * 16, 16)] + w0j * hq)

        def cs(jb, c):
            j0 = jb * 2
            have2 = (j0 + 1) < cnt_own
            da = pltpu.async_copy(agg_sh.at[nb + j0], arow, sem2)

            @pl.when(have2)
            def _():
                pltpu.async_copy(agg_sh.at[nb + j0 + 1], arow2, sem2)
            da.wait()
            consume_row(arow, j0)

            @pl.when(have2)
            def _():
                pltpu.make_async_copy(agg_sh.at[nb + j0 + 1], arow2,
                                      sem2).wait()
                consume_row(arow2, j0 + 1)
            return c
        lax.fori_loop(0, (cnt_own + 1) >> 1, cs, 0)
        return carry
    lax.fori_loop(0, _T, t_loop, 0)

    pltpu.sync_copy(u_loc, up_hbm.at[pl.ds(pl.multiple_of(tid * _T * _H, 8), _T * _H)])


def _graph_call(s, d, ew, x2d, w1f, b1):
    return pl.kernel(
        _graph_body,
        out_type=(
            jax.ShapeDtypeStruct((_NT * _T * _H,), jnp.float32),
            jax.ShapeDtypeStruct((_NT * _NPAD,), jnp.float32),
            jax.ShapeDtypeStruct((_NT * _NPAD,), jnp.float32),
            jax.ShapeDtypeStruct((_NT * _ROW,), jnp.int32),
            jax.ShapeDtypeStruct((_NT * _ROW,), jnp.int32),
            jax.ShapeDtypeStruct((_NT * _ROW,), jnp.float32),
        ),
        mesh=plsc.VectorSubcoreMesh(core_axis_name="c", subcore_axis_name="s",
                                    num_cores=1),
        compiler_params=pltpu.CompilerParams(needs_layout_passes=False),
        scratch_types=[
            pltpu.VMEM((_CH,), jnp.int32),
            pltpu.VMEM((_CH,), jnp.int32),
            pltpu.VMEM((_CH,), jnp.float32),
            pltpu.VMEM((_NPAD,), jnp.float32),
            pltpu.VMEM((_NPAD,), jnp.float32),
            pltpu.VMEM((_NPT,), jnp.float32),
            pltpu.VMEM((_NPT,), jnp.float32),
            pltpu.VMEM((_NPT,), jnp.float32),
            pltpu.VMEM((_NPT + 16,), jnp.float32),
            pltpu.VMEM((4, _D), jnp.float32),
            pltpu.VMEM((_STG,), jnp.int32),
            pltpu.VMEM((_STG,), jnp.int32),
            pltpu.VMEM((_STG,), jnp.float32),
            pltpu.VMEM((_D * _H,), jnp.float32),
            pltpu.VMEM((_H,), jnp.float32),
            pltpu.VMEM((32,), jnp.int32),
            pltpu.VMEM((16,), jnp.int32),
            pltpu.VMEM((32, _D), jnp.float32),
            pltpu.VMEM((_D,), jnp.float32),
            pltpu.VMEM((_D,), jnp.float32),
            pltpu.VMEM((_T * _H,), jnp.float32),
            pltpu.VMEM_SHARED((_NPAD,), jnp.float32),
            pltpu.VMEM_SHARED((_NPAD,), jnp.float32),
            pltpu.VMEM_SHARED((_AGG_ROWS, _D), jnp.float32),
            pltpu.SemaphoreType.DMA,
            pltpu.SemaphoreType.DMA,
        ],
    )(s, d, ew, x2d, w1f, b1)


def _head_body(m_ref, up_ref, w2_ref, b2_ref, wih_ref, whh_ref,
               bih_ref, bhh_ref, hwt_ref, hb_ref, out_ref):
    u = jnp.dot(m_ref[...], up_ref[...], preferred_element_type=jnp.float32)
    seq = jnp.maximum(
        jnp.dot(u, w2_ref[...], preferred_element_type=jnp.float32)
        + b2_ref[...][None, :], 0.0)
    h = jnp.zeros((1, _H), jnp.float32)
    for t in range(_T):
        xt = seq[t:t + 1, :]
        gi = jnp.dot(xt, wih_ref[...],
                     preferred_element_type=jnp.float32) + bih_ref[...][None, :]
        gh = jnp.dot(h, whh_ref[...],
                     preferred_element_type=jnp.float32) + bhh_ref[...][None, :]
        r = jax.nn.sigmoid(gi[:, :_H] + gh[:, :_H])
        z = jax.nn.sigmoid(gi[:, _H:2 * _H] + gh[:, _H:2 * _H])
        n = jnp.tanh(gi[:, 2 * _H:] + r * gh[:, 2 * _H:])
        h = (1.0 - z) * n + z * h
    logits = jnp.dot(h, hwt_ref[...],
                     preferred_element_type=jnp.float32) + hb_ref[...][None, :]
    out_ref[...] = jax.nn.sigmoid(logits)


def _head_call(m, up, W2, b2, wihT, whhT, b_ih, b_hh, hWT, heads_b):
    return pl.pallas_call(
        _head_body,
        out_shape=jax.ShapeDtypeStruct((1, 8), jnp.float32),
    )(m, up, W2, b2, wihT, whhT, b_ih, b_hh, hWT, heads_b)


def kernel(x, edge_index, edge_weight, W1, b1, W2, b2, W_ih, W_hh,
           b_ih, b_hh, heads_W, heads_b):
    s = edge_index[0].astype(jnp.int32)
    d = edge_index[1].astype(jnp.int32)
    x2d = x.reshape(_T * _N, _D)
    outs = _graph_call(s, d, edge_weight.astype(jnp.float32), x2d,
                       W1.reshape(-1), b1)
    up = outs[0].reshape(_NT * _T, _H)
    m = jnp.tile(jnp.eye(_T, dtype=jnp.float32), (1, _NT))
    return _head_call(m, up, W2, b2, W_ih.T, W_hh.T, b_ih, b_hh,
                      heads_W.T, heads_b)


# speedup vs baseline: 294.3865x; 1.0487x over previous
import jax
import jax.numpy as jnp
from jax import lax
from jax.experimental import pallas as pl
from jax.experimental.pallas import tpu as pltpu
from jax.experimental.pallas import tpu_sc as plsc

_T, _N, _D, _H = 12, 10000, 128, 64
_E = 320000
_NT = 16
_NPT = 640
_NPAD = _NT * _NPT
_EPT = _E // _NT
_CH = 800
_HC = _CH // 2
_NH = _EPT // _HC
_NPAIR = _NH // 2
_STG = 2112
_ROW = 22528
_DUMP = _NPAD
_AGG_ROWS = _NPAD + 8


def _graph_body(s_hbm, d_hbm, ew_hbm, x_hbm, w1_hbm, b1_hbm,
                up_hbm, degp_hbm, c0p_hbm, selS_hbm, selT_hbm, selW_hbm,
                s_ch, d_ch, ew_ch,
                nodef_a, nodef_b,
                degsl, c0sl, invsl, w0cmp, z4,
                stg_s, stg_t, stg_w,
                w1_loc, b1_loc,
                gidx4, sidx, sidx2, xrowA, arow, arow2, u_loc,
                dinv_sh, inv_sh, agg_sh,
                sem, sem2, semA, semB):
    tid = lax.axis_index("s")
    i16 = lax.iota(jnp.int32, 16)
    zf = jnp.zeros((16,), jnp.float32)
    nb = tid * _NPT

    def z_big(i, c):
        nodef_a[pl.ds(i * 16, 16)] = zf
        nodef_b[pl.ds(i * 16, 16)] = zf
        return c
    lax.fori_loop(0, _NPAD // 16, z_big, 0, unroll=4)

    def z_small(i, c):
        degsl[pl.ds(i * 16, 16)] = zf
        c0sl[pl.ds(i * 16, 16)] = zf
        return c
    lax.fori_loop(0, _NPT // 16, z_small, 0)

    def z_u(i, c):
        u_loc[pl.ds(i * 16, 16)] = zf
        return c
    lax.fori_loop(0, (_T * _H) // 16, z_u, 0)
    for r in range(4):
        for q in range(_D // 16):
            z4[r, pl.ds(q * 16, 16)] = zf

    pltpu.sync_copy(w1_hbm, w1_loc)
    pltpu.sync_copy(b1_hbm, b1_loc)

    def _issue_half(h, lo, bsem):
        base = pl.multiple_of(tid * _EPT + h * _HC, 8)
        pltpu.async_copy(s_hbm.at[pl.ds(base, _HC)],
                         s_ch.at[pl.ds(lo, _HC)], bsem)
        pltpu.async_copy(d_hbm.at[pl.ds(base, _HC)],
                         d_ch.at[pl.ds(lo, _HC)], bsem)
        pltpu.async_copy(ew_hbm.at[pl.ds(base, _HC)],
                         ew_ch.at[pl.ds(lo, _HC)], bsem)

    def _drain_half(lo, bsem):
        pltpu.make_async_copy(s_hbm.at[pl.ds(0, _HC)],
                              s_ch.at[pl.ds(lo, _HC)], bsem).wait()
        pltpu.make_async_copy(d_hbm.at[pl.ds(0, _HC)],
                              d_ch.at[pl.ds(lo, _HC)], bsem).wait()
        pltpu.make_async_copy(ew_hbm.at[pl.ds(0, _HC)],
                              ew_ch.at[pl.ds(lo, _HC)], bsem).wait()

    def p1_process(lo):
        def p1_v(i, cc):
            sv = s_ch[pl.ds(lo + i * 16, 16)]
            dv = d_ch[pl.ds(lo + i * 16, 16)]
            ev = ew_ch[pl.ds(lo + i * 16, 16)]
            plsc.addupdate_scatter(nodef_a, [dv], ev)
            plsc.addupdate_scatter(nodef_b, [sv],
                                   jnp.where(dv == 0, ev, jnp.float32(0.0)))
            return cc
        lax.fori_loop(0, _HC // 16, p1_v, 0, unroll=4)

    _issue_half(0, 0, semA)
    def p1_pair(hp, carry):
        h0 = hp * 2
        _drain_half(0, semA)
        _issue_half(h0 + 1, _HC, semB)
        p1_process(0)
        _drain_half(_HC, semB)

        @pl.when(hp < _NPAIR - 1)
        def _():
            _issue_half(h0 + 2, 0, semA)
        p1_process(_HC)
        return carry
    lax.fori_loop(0, _NPAIR, p1_pair, 0)

    descs = []
    for i in range(_NT):
        dsto = pl.multiple_of(i * _NPAD + tid * _NPT, 8)
        descs.append(pltpu.async_copy(
            nodef_a.at[pl.ds(i * _NPT, _NPT)],
            degp_hbm.at[pl.ds(dsto, _NPT)], sem))
        descs.append(pltpu.async_copy(
            nodef_b.at[pl.ds(i * _NPT, _NPT)],
            c0p_hbm.at[pl.ds(dsto, _NPT)], sem))
    for dsc in descs:
        dsc.wait()
    plsc.subcore_barrier()
    pltpu.sync_copy(degp_hbm.at[pl.ds(pl.multiple_of(tid * _NPAD, 8), _NPAD)],
                    nodef_a)
    pltpu.sync_copy(c0p_hbm.at[pl.ds(pl.multiple_of(tid * _NPAD, 8), _NPAD)],
                    nodef_b)
    for j in range(_NT):
        def accd(i, c):
            degsl[pl.ds(i * 16, 16)] = (degsl[pl.ds(i * 16, 16)]
                                        + nodef_a[pl.ds(j * _NPT + i * 16, 16)])
            c0sl[pl.ds(i * 16, 16)] = (c0sl[pl.ds(i * 16, 16)]
                                       + nodef_b[pl.ds(j * _NPT + i * 16, 16)])
            return c
        lax.fori_loop(0, _NPT // 16, accd, 0, unroll=4)

    def p3(i, c):
        dg = degsl[pl.ds(i * 16, 16)] + 1.0
        ib = plsc.bitcast(dg, jnp.int32)
        ib = 0x5F3759DF - (ib >> 1)
        y = plsc.bitcast(ib, jnp.float32)
        y = y * (1.5 - 0.5 * dg * y * y)
        y = y * (1.5 - 0.5 * dg * y * y)
        y = y * (1.5 - 0.5 * dg * y * y)
        degsl[pl.ds(i * 16, 16)] = y
        return c
    lax.fori_loop(0, _NPT // 16, p3, 0)
    pltpu.sync_copy(degsl, dinv_sh.at[pl.ds(pl.multiple_of(nb, 8), _NPT)])
    plsc.subcore_barrier()
    pltpu.sync_copy(dinv_sh, nodef_a)

    def _flush(off, cnt):
        for p in range(4):
            pos = cnt + p * 16 + i16
            plsc.store_scatter(stg_s, [pos], jnp.zeros((16,), jnp.int32))
            plsc.store_scatter(stg_t, [pos], jnp.full((16,), _DUMP, jnp.int32))
            plsc.store_scatter(stg_w, [pos], zf)
        padded = ((cnt + 63) >> 6) << 6

        def fl(p, c):
            dst = pl.multiple_of(tid * _ROW + off + p * 64, 8)
            pltpu.sync_copy(stg_s.at[pl.ds(p * 64, 64)],
                            selS_hbm.at[pl.ds(dst, 64)])
            pltpu.sync_copy(stg_t.at[pl.ds(p * 64, 64)],
                            selT_hbm.at[pl.ds(dst, 64)])
            pltpu.sync_copy(stg_w.at[pl.ds(p * 64, 64)],
                            selW_hbm.at[pl.ds(dst, 64)])
            return c
        lax.fori_loop(0, padded >> 6, fl, 0)
        return off + padded

    dinv0 = nodef_a[pl.ds(0, 16)][0]

    def p4(i, st):
        lslot = st
        gid = nb + i * 16 + i16
        c0v = c0sl[pl.ds(i * 16, 16)]
        dvv = degsl[pl.ds(i * 16, 16)]
        m = (c0v > 0.0) | (gid == 0)
        mi = m.astype(jnp.int32)
        excl = plsc.cumsum(mi) - mi
        cnt = jnp.sum(mi)
        slot_local = lslot + excl
        w0v = dinv0 * (c0v * dvv + jnp.where(gid == 0, dinv0, jnp.float32(0.0)))
        plsc.store_scatter(w0cmp, [slot_local], w0v, mask=m)
        invsl[pl.ds(i * 16, 16)] = jnp.where(
            m, (nb + slot_local + 1).astype(jnp.float32), jnp.float32(0.0))
        plsc.store_scatter(stg_s, [slot_local], gid, mask=m)
        plsc.store_scatter(stg_t, [slot_local], nb + slot_local, mask=m)
        plsc.store_scatter(stg_w, [slot_local], dvv * dvv, mask=m)
        return lslot + cnt
    cnt_own = lax.fori_loop(0, _NPT // 16, p4, jnp.int32(0))
    pltpu.sync_copy(invsl, inv_sh.at[pl.ds(pl.multiple_of(nb, 8), _NPT)])
    plsc.subcore_barrier()
    pltpu.sync_copy(inv_sh, nodef_b)

    def p5_process(lo, carry):
        off, lc0 = carry

        def p5_v(i, lc):
            sv = s_ch[pl.ds(lo + i * 16, 16)]
            dv = d_ch[pl.ds(lo + i * 16, 16)]
            ev = ew_ch[pl.ds(lo + i * 16, 16)]
            g = plsc.load_gather(nodef_b, [dv])
            m = g > 0.5
            mi = m.astype(jnp.int32)
            app = lc + plsc.cumsum(mi) - mi
            nrm = (plsc.load_gather(nodef_a, [sv]) * ev *
                   plsc.load_gather(nodef_a, [dv]))
            plsc.store_scatter(stg_s, [app], sv, mask=m)
            plsc.store_scatter(stg_t, [app], (g - 1.0).astype(jnp.int32),
                               mask=m)
            plsc.store_scatter(stg_w, [app], nrm, mask=m)
            return lc + jnp.sum(mi)
        lc = lax.fori_loop(0, _HC // 16, p5_v, lc0, unroll=2)

        def spill(args):
            o, l = args
            for p in range(16):
                dst = pl.multiple_of(tid * _ROW + o + p * 64, 8)
                pltpu.sync_copy(stg_s.at[pl.ds(p * 64, 64)],
                                selS_hbm.at[pl.ds(dst, 64)])
                pltpu.sync_copy(stg_t.at[pl.ds(p * 64, 64)],
                                selT_hbm.at[pl.ds(dst, 64)])
                pltpu.sync_copy(stg_w.at[pl.ds(p * 64, 64)],
                                selW_hbm.at[pl.ds(dst, 64)])

            def sh(i, cc):
                stg_s[pl.ds(i * 16, 16)] = stg_s[pl.ds(1024 + i * 16, 16)]
                stg_t[pl.ds(i * 16, 16)] = stg_t[pl.ds(1024 + i * 16, 16)]
                stg_w[pl.ds(i * 16, 16)] = stg_w[pl.ds(1024 + i * 16, 16)]
                return cc
            lax.fori_loop(0, (l - 1024 + 15) >> 4, sh, 0)
            return (o + 1024, l - 1024)
        return lax.cond(lc >= 1024, spill, lambda a: a, (off, lc))

    _issue_half(0, 0, semA)
    def p5_pair(hp, carry):
        h0 = hp * 2
        _drain_half(0, semA)
        _issue_half(h0 + 1, _HC, semB)
        carry = p5_process(0, carry)
        _drain_half(_HC, semB)

        @pl.when(hp < _NPAIR - 1)
        def _():
            _issue_half(h0 + 2, 0, semA)
        carry = p5_process(_HC, carry)
        return carry
    off, lc = lax.fori_loop(0, _NPAIR, p5_pair, (jnp.int32(0), cnt_own))
    off = _flush(off, lc)
    ngroups = off >> 4

    nch = (off + 1023) >> 10

    def load_sel_chunk(ci):
        cbase = pl.multiple_of(tid * _ROW + ci * 1024, 8)
        d0 = pltpu.async_copy(selS_hbm.at[pl.ds(cbase, 1024)],
                              stg_s.at[pl.ds(0, 1024)], sem)
        d1 = pltpu.async_copy(selT_hbm.at[pl.ds(cbase, 1024)],
                              stg_t.at[pl.ds(0, 1024)], sem)
        d2 = pltpu.async_copy(selW_hbm.at[pl.ds(cbase, 1024)],
                              stg_w.at[pl.ds(0, 1024)], sem)
        d0.wait(); d1.wait(); d2.wait()
    load_sel_chunk(0)

    def t_loop(t, carry):
        nzb = (cnt_own + 3) >> 2

        def za(jb, c):
            pltpu.async_copy(z4, agg_sh.at[pl.ds(nb + jb * 4, 4)], semA)
            return c
        lax.fori_loop(0, nzb, za, 0)

        @pl.when(jnp.logical_and(nch == 1, ngroups > 0))
        def _():
            for b in range(2):
                gidx4[pl.ds(b * 16, 16)] = (
                    stg_s[pl.ds(b * 16, 16)] + t * _N)
            pltpu.async_copy(x_hbm.at[gidx4.at[pl.ds(0, 16)]],
                             xrowA.at[pl.ds(0, 16)], sem2)
            pltpu.async_copy(x_hbm.at[gidx4.at[pl.ds(16, 16)]],
                             xrowA.at[pl.ds(16, 16)], sem2)

        def zwait(jb, c):
            pltpu.make_async_copy(z4, agg_sh.at[pl.ds(nb + jb * 4, 4)],
                                  semA).wait()
            return c
        lax.fori_loop(0, nzb, zwait, 0)
        plsc.subcore_barrier()

        def cchunk(ci, c):
            @pl.when(jnp.logical_or(ci > 0, nch > 1))
            def _():
                load_sel_chunk(ci)
            ng_here = jnp.minimum(64, ngroups - ci * 64)

            def blk(bi, cc):
                g0 = bi * 2

                @pl.when(jnp.logical_or(bi > 0, nch > 1))
                def _():
                    for b in range(2):
                        gidx4[pl.ds(b * 16, 16)] = (
                            stg_s[pl.ds((g0 + b) * 16, 16)] + t * _N)
                    pltpu.async_copy(x_hbm.at[gidx4.at[pl.ds(0, 16)]],
                                     xrowA.at[pl.ds(0, 16)], sem2)
                    pltpu.async_copy(x_hbm.at[gidx4.at[pl.ds(16, 16)]],
                                     xrowA.at[pl.ds(16, 16)], sem2)
                pltpu.make_async_copy(x_hbm.at[gidx4.at[pl.ds(0, 16)]],
                                      xrowA.at[pl.ds(0, 16)], sem2).wait()
                pltpu.make_async_copy(x_hbm.at[gidx4.at[pl.ds(16, 16)]],
                                      xrowA.at[pl.ds(16, 16)], sem2).wait()
                for b in range(2):
                    nv = stg_w[pl.ds((g0 + b) * 16, 16)]
                    for r in range(16):
                        nrm = nv[r]
                        for q in range(_D // 16):
                            xrowA[b * 16 + r, pl.ds(q * 16, 16)] = (
                                xrowA[b * 16 + r, pl.ds(q * 16, 16)] * nrm)
                    if b == 0:
                        sidx[...] = stg_t[pl.ds(g0 * 16, 16)]
                        pltpu.async_copy(xrowA.at[pl.ds(0, 16)],
                                         agg_sh.at[sidx], semA, add=True)
                    else:
                        sidx2[...] = stg_t[pl.ds((g0 + 1) * 16, 16)]
                        pltpu.async_copy(xrowA.at[pl.ds(16, 16)],
                                         agg_sh.at[sidx2], semA, add=True)
                pltpu.make_async_copy(xrowA.at[pl.ds(0, 16)],
                                      agg_sh.at[sidx], semA).wait()
                pltpu.make_async_copy(xrowA.at[pl.ds(16, 16)],
                                      agg_sh.at[sidx2], semA).wait()
                return cc
            lax.fori_loop(0, ng_here >> 1, blk, 0)
            return c
        lax.fori_loop(0, nch, cchunk, 0)
        plsc.subcore_barrier()

        def consume_row(buf, j):
            w0j = w0cmp[pl.ds(j, 16)][0]

            def kk(kb, acc):
                a0, a1, a2, a3 = acc
                av = buf[pl.ds(kb * 16, 16)]
                for r in range(16):
                    ak = av[r]
                    k = kb * 16 + r
                    a0 = a0 + ak * w1_loc[pl.ds(k * 64, 16)]
                    a1 = a1 + ak * w1_loc[pl.ds(k * 64 + 16, 16)]
                    a2 = a2 + ak * w1_loc[pl.ds(k * 64 + 32, 16)]
                    a3 = a3 + ak * w1_loc[pl.ds(k * 64 + 48, 16)]
                return (a0, a1, a2, a3)
            accs = lax.fori_loop(0, _D // 16, kk, (zf, zf, zf, zf))
            for q in range(4):
                hq = jnp.maximum(accs[q] + b1_loc[pl.ds(q * 16, 16)], 0.0)
                u_loc[pl.ds(t * 64 + q * 16, 16)] = (
                    u_loc[pl.ds(t * 64 + q * 16, 16)] + w0j * hq)

        def cs(jb, c):
            j0 = jb * 2
            have2 = (j0 + 1) < cnt_own
            da = pltpu.async_copy(agg_sh.at[nb + j0], arow, sem2)

            @pl.when(have2)
            def _():
                pltpu.async_copy(agg_sh.at[nb + j0 + 1], arow2, sem2)
            da.wait()
            consume_row(arow, j0)

            @pl.when(have2)
            def _():
                pltpu.make_async_copy(agg_sh.at[nb + j0 + 1], arow2,
                                      sem2).wait()
                consume_row(arow2, j0 + 1)
            return c
        lax.fori_loop(0, (cnt_own + 1) >> 1, cs, 0)
        return carry
    lax.fori_loop(0, _T, t_loop, 0)

    pltpu.sync_copy(u_loc, up_hbm.at[pl.ds(pl.multiple_of(tid * _T * _H, 8), _T * _H)])


def _graph_call(s, d, ew, x2d, w1f, b1):
    return pl.kernel(
        _graph_body,
        out_type=(
            jax.ShapeDtypeStruct((_NT * _T * _H,), jnp.float32),
            jax.ShapeDtypeStruct((_NT * _NPAD,), jnp.float32),
            jax.ShapeDtypeStruct((_NT * _NPAD,), jnp.float32),
            jax.ShapeDtypeStruct((_NT * _ROW,), jnp.int32),
            jax.ShapeDtypeStruct((_NT * _ROW,), jnp.int32),
            jax.ShapeDtypeStruct((_NT * _ROW,), jnp.float32),
        ),
        mesh=plsc.VectorSubcoreMesh(core_axis_name="c", subcore_axis_name="s",
                                    num_cores=1),
        compiler_params=pltpu.CompilerParams(needs_layout_passes=False),
        scratch_types=[
            pltpu.VMEM((_CH,), jnp.int32),
            pltpu.VMEM((_CH,), jnp.int32),
            pltpu.VMEM((_CH,), jnp.float32),
            pltpu.VMEM((_NPAD,), jnp.float32),
            pltpu.VMEM((_NPAD,), jnp.float32),
            pltpu.VMEM((_NPT,), jnp.float32),
            pltpu.VMEM((_NPT,), jnp.float32),
            pltpu.VMEM((_NPT,), jnp.float32),
            pltpu.VMEM((_NPT + 16,), jnp.float32),
            pltpu.VMEM((4, _D), jnp.float32),
            pltpu.VMEM((_STG,), jnp.int32),
            pltpu.VMEM((_STG,), jnp.int32),
            pltpu.VMEM((_STG,), jnp.float32),
            pltpu.VMEM((_D * _H,), jnp.float32),
            pltpu.VMEM((_H,), jnp.float32),
            pltpu.VMEM((32,), jnp.int32),
            pltpu.VMEM((16,), jnp.int32),
            pltpu.VMEM((16,), jnp.int32),
            pltpu.VMEM((32, _D), jnp.float32),
            pltpu.VMEM((_D,), jnp.float32),
            pltpu.VMEM((_D,), jnp.float32),
            pltpu.VMEM((_T * _H,), jnp.float32),
            pltpu.VMEM_SHARED((_NPAD,), jnp.float32),
            pltpu.VMEM_SHARED((_NPAD,), jnp.float32),
            pltpu.VMEM_SHARED((_AGG_ROWS, _D), jnp.float32),
            pltpu.SemaphoreType.DMA,
            pltpu.SemaphoreType.DMA,
            pltpu.SemaphoreType.DMA,
            pltpu.SemaphoreType.DMA,
        ],
    )(s, d, ew, x2d, w1f, b1)


def _head_body(m_ref, up_ref, w2_ref, b2_ref, wih_ref, whh_ref,
               bih_ref, bhh_ref, hwt_ref, hb_ref, out_ref):
    u = jnp.dot(m_ref[...], up_ref[...], preferred_element_type=jnp.float32)
    seq = jnp.maximum(
        jnp.dot(u, w2_ref[...], preferred_element_type=jnp.float32)
        + b2_ref[...][None, :], 0.0)
    h = jnp.zeros((1, _H), jnp.float32)
    for t in range(_T):
        xt = seq[t:t + 1, :]
        gi = jnp.dot(xt, wih_ref[...],
                     preferred_element_type=jnp.float32) + bih_ref[...][None, :]
        gh = jnp.dot(h, whh_ref[...],
                     preferred_element_type=jnp.float32) + bhh_ref[...][None, :]
        r = jax.nn.sigmoid(gi[:, :_H] + gh[:, :_H])
        z = jax.nn.sigmoid(gi[:, _H:2 * _H] + gh[:, _H:2 * _H])
        n = jnp.tanh(gi[:, 2 * _H:] + r * gh[:, 2 * _H:])
        h = (1.0 - z) * n + z * h
    logits = jnp.dot(h, hwt_ref[...],
                     preferred_element_type=jnp.float32) + hb_ref[...][None, :]
    out_ref[...] = jax.nn.sigmoid(logits)


def _head_call(m, up, W2, b2, wihT, whhT, b_ih, b_hh, hWT, heads_b):
    return pl.pallas_call(
        _head_body,
        out_shape=jax.ShapeDtypeStruct((1, 8), jnp.float32),
    )(m, up, W2, b2, wihT, whhT, b_ih, b_hh, hWT, heads_b)


def kernel(x, edge_index, edge_weight, W1, b1, W2, b2, W_ih, W_hh,
           b_ih, b_hh, heads_W, heads_b):
    s = edge_index[0].astype(jnp.int32)
    d = edge_index[1].astype(jnp.int32)
    x2d = x.reshape(_T * _N, _D)
    outs = _graph_call(s, d, edge_weight.astype(jnp.float32), x2d,
                       W1.reshape(-1), b1)
    up = outs[0].reshape(_NT * _T, _H)
    m = jnp.tile(jnp.eye(_T, dtype=jnp.float32), (1, _NT))
    return _head_call(m, up, W2, b2, W_ih.T, W_hh.T, b_ih, b_hh,
                      heads_W.T, heads_b)


# speedup vs baseline: 295.5121x; 1.0038x over previous
import jax
import jax.numpy as jnp
from jax import lax
from jax.experimental import pallas as pl
from jax.experimental.pallas import tpu as pltpu
from jax.experimental.pallas import tpu_sc as plsc

_T, _N, _D, _H = 12, 10000, 128, 64
_E = 320000
_NT = 16
_NPT = 640
_NPAD = _NT * _NPT
_EPT = _E // _NT
_CH = 800
_HC = _CH // 2
_NH = _EPT // _HC
_NPAIR = _NH // 2
_STG = 2112
_ROW = 22528
_DUMP = _NPAD
_AGG_ROWS = _NPAD + 8


def _graph_body(s_hbm, d_hbm, ew_hbm, x_hbm, w1_hbm, b1_hbm,
                up_hbm, degp_hbm, c0p_hbm, selS_hbm, selT_hbm, selW_hbm,
                s_ch, d_ch, ew_ch,
                nodef_a, nodef_b,
                degsl, c0sl, invsl, w0cmp, z4,
                stg_s, stg_t, stg_w,
                w1_loc, b1_loc,
                gidx4, sidx, sidx2, xrowA, arow, arow2, u_loc,
                dinv_sh, inv_sh, agg_sh,
                sem, sem2, semA, semB):
    tid = lax.axis_index("s")
    i16 = lax.iota(jnp.int32, 16)
    zf = jnp.zeros((16,), jnp.float32)
    nb = tid * _NPT

    def z_big(i, c):
        nodef_a[pl.ds(i * 16, 16)] = zf
        nodef_b[pl.ds(i * 16, 16)] = zf
        return c
    lax.fori_loop(0, _NPAD // 16, z_big, 0, unroll=4)

    def z_small(i, c):
        degsl[pl.ds(i * 16, 16)] = zf
        c0sl[pl.ds(i * 16, 16)] = zf
        return c
    lax.fori_loop(0, _NPT // 16, z_small, 0)

    def z_u(i, c):
        u_loc[pl.ds(i * 16, 16)] = zf
        return c
    lax.fori_loop(0, (_T * _H) // 16, z_u, 0)
    for r in range(4):
        for q in range(_D // 16):
            z4[r, pl.ds(q * 16, 16)] = zf

    pltpu.sync_copy(w1_hbm, w1_loc)
    pltpu.sync_copy(b1_hbm, b1_loc)

    def _issue_half(h, lo, bsem):
        base = pl.multiple_of(tid * _EPT + h * _HC, 8)
        pltpu.async_copy(s_hbm.at[pl.ds(base, _HC)],
                         s_ch.at[pl.ds(lo, _HC)], bsem)
        pltpu.async_copy(d_hbm.at[pl.ds(base, _HC)],
                         d_ch.at[pl.ds(lo, _HC)], bsem)
        pltpu.async_copy(ew_hbm.at[pl.ds(base, _HC)],
                         ew_ch.at[pl.ds(lo, _HC)], bsem)

    def _drain_half(lo, bsem):
        pltpu.make_async_copy(s_hbm.at[pl.ds(0, _HC)],
                              s_ch.at[pl.ds(lo, _HC)], bsem).wait()
        pltpu.make_async_copy(d_hbm.at[pl.ds(0, _HC)],
                              d_ch.at[pl.ds(lo, _HC)], bsem).wait()
        pltpu.make_async_copy(ew_hbm.at[pl.ds(0, _HC)],
                              ew_ch.at[pl.ds(lo, _HC)], bsem).wait()

    def p1_process(lo):
        def p1_v(i, cc):
            sv = s_ch[pl.ds(lo + i * 16, 16)]
            dv = d_ch[pl.ds(lo + i * 16, 16)]
            ev = ew_ch[pl.ds(lo + i * 16, 16)]
            plsc.addupdate_scatter(nodef_a, [dv], ev)
            plsc.addupdate_scatter(nodef_b, [sv],
                                   jnp.where(dv == 0, ev, jnp.float32(0.0)))
            return cc
        lax.fori_loop(0, _HC // 16, p1_v, 0, unroll=4)

    _issue_half(0, 0, semA)
    def p1_pair(hp, carry):
        h0 = hp * 2
        _drain_half(0, semA)
        _issue_half(h0 + 1, _HC, semB)
        p1_process(0)
        _drain_half(_HC, semB)

        @pl.when(hp < _NPAIR - 1)
        def _():
            _issue_half(h0 + 2, 0, semA)
        p1_process(_HC)
        return carry
    lax.fori_loop(0, _NPAIR, p1_pair, 0)

    descs = []
    for i in range(_NT):
        dsto = pl.multiple_of(i * _NPAD + tid * _NPT, 8)
        descs.append(pltpu.async_copy(
            nodef_a.at[pl.ds(i * _NPT, _NPT)],
            degp_hbm.at[pl.ds(dsto, _NPT)], sem))
        descs.append(pltpu.async_copy(
            nodef_b.at[pl.ds(i * _NPT, _NPT)],
            c0p_hbm.at[pl.ds(dsto, _NPT)], sem))
    for dsc in descs:
        dsc.wait()
    plsc.subcore_barrier()
    pltpu.sync_copy(degp_hbm.at[pl.ds(pl.multiple_of(tid * _NPAD, 8), _NPAD)],
                    nodef_a)
    pltpu.sync_copy(c0p_hbm.at[pl.ds(pl.multiple_of(tid * _NPAD, 8), _NPAD)],
                    nodef_b)
    for j in range(_NT):
        def accd(i, c):
            degsl[pl.ds(i * 16, 16)] = (degsl[pl.ds(i * 16, 16)]
                                        + nodef_a[pl.ds(j * _NPT + i * 16, 16)])
            c0sl[pl.ds(i * 16, 16)] = (c0sl[pl.ds(i * 16, 16)]
                                       + nodef_b[pl.ds(j * _NPT + i * 16, 16)])
            return c
        lax.fori_loop(0, _NPT // 16, accd, 0, unroll=4)

    def p3(i, c):
        dg = degsl[pl.ds(i * 16, 16)] + 1.0
        ib = plsc.bitcast(dg, jnp.int32)
        ib = 0x5F3759DF - (ib >> 1)
        y = plsc.bitcast(ib, jnp.float32)
        y = y * (1.5 - 0.5 * dg * y * y)
        y = y * (1.5 - 0.5 * dg * y * y)
        y = y * (1.5 - 0.5 * dg * y * y)
        degsl[pl.ds(i * 16, 16)] = y
        return c
    lax.fori_loop(0, _NPT // 16, p3, 0)
    pltpu.sync_copy(degsl, dinv_sh.at[pl.ds(pl.multiple_of(nb, 8), _NPT)])
    plsc.subcore_barrier()
    pltpu.sync_copy(dinv_sh, nodef_a)

    def _flush(off, cnt):
        for p in range(4):
            pos = cnt + p * 16 + i16
            plsc.store_scatter(stg_s, [pos], jnp.zeros((16,), jnp.int32))
            plsc.store_scatter(stg_t, [pos], jnp.full((16,), _DUMP, jnp.int32))
            plsc.store_scatter(stg_w, [pos], zf)
        padded = ((cnt + 63) >> 6) << 6

        def fl(p, c):
            dst = pl.multiple_of(tid * _ROW + off + p * 64, 8)
            pltpu.sync_copy(stg_s.at[pl.ds(p * 64, 64)],
                            selS_hbm.at[pl.ds(dst, 64)])
            pltpu.sync_copy(stg_t.at[pl.ds(p * 64, 64)],
                            selT_hbm.at[pl.ds(dst, 64)])
            pltpu.sync_copy(stg_w.at[pl.ds(p * 64, 64)],
                            selW_hbm.at[pl.ds(dst, 64)])
            return c
        lax.fori_loop(0, padded >> 6, fl, 0)
        return off + padded

    dinv0 = nodef_a[pl.ds(0, 16)][0]

    def p4(i, st):
        lslot = st
        gid = nb + i * 16 + i16
        c0v = c0sl[pl.ds(i * 16, 16)]
        dvv = degsl[pl.ds(i * 16, 16)]
        m = (c0v > 0.0) | (gid == 0)
        mi = m.astype(jnp.int32)
        excl = plsc.cumsum(mi) - mi
        cnt = jnp.sum(mi)
        slot_local = lslot + excl
        w0v = dinv0 * (c0v * dvv + jnp.where(gid == 0, dinv0, jnp.float32(0.0)))
        plsc.store_scatter(w0cmp, [slot_local], w0v, mask=m)
        invsl[pl.ds(i * 16, 16)] = jnp.where(
            m, (nb + slot_local + 1).astype(jnp.float32), jnp.float32(0.0))
        plsc.store_scatter(stg_s, [slot_local], gid, mask=m)
        plsc.store_scatter(stg_t, [slot_local], nb + slot_local, mask=m)
        plsc.store_scatter(stg_w, [slot_local], dvv * dvv, mask=m)
        return lslot + cnt
    cnt_own = lax.fori_loop(0, _NPT // 16, p4, jnp.int32(0))
    pltpu.sync_copy(invsl, inv_sh.at[pl.ds(pl.multiple_of(nb, 8), _NPT)])
    plsc.subcore_barrier()
    pltpu.sync_copy(inv_sh, nodef_b)

    def p5_process(lo, carry):
        off, lc0 = carry

        def p5_v(i, lc):
            sv = s_ch[pl.ds(lo + i * 16, 16)]
            dv = d_ch[pl.ds(lo + i * 16, 16)]
            ev = ew_ch[pl.ds(lo + i * 16, 16)]
            g = plsc.load_gather(nodef_b, [dv])
            m = g > 0.5
            mi = m.astype(jnp.int32)
            app = lc + plsc.cumsum(mi) - mi
            nrm = (plsc.load_gather(nodef_a, [sv]) * ev *
                   plsc.load_gather(nodef_a, [dv]))
            plsc.store_scatter(stg_s, [app], sv, mask=m)
            plsc.store_scatter(stg_t, [app], (g - 1.0).astype(jnp.int32),
                               mask=m)
            plsc.store_scatter(stg_w, [app], nrm, mask=m)
            return lc + jnp.sum(mi)
        lc = lax.fori_loop(0, _HC // 16, p5_v, lc0, unroll=2)

        def spill(args):
            o, l = args
            for p in range(16):
                dst = pl.multiple_of(tid * _ROW + o + p * 64, 8)
                pltpu.sync_copy(stg_s.at[pl.ds(p * 64, 64)],
                                selS_hbm.at[pl.ds(dst, 64)])
                pltpu.sync_copy(stg_t.at[pl.ds(p * 64, 64)],
                                selT_hbm.at[pl.ds(dst, 64)])
                pltpu.sync_copy(stg_w.at[pl.ds(p * 64, 64)],
                                selW_hbm.at[pl.ds(dst, 64)])

            def sh(i, cc):
                stg_s[pl.ds(i * 16, 16)] = stg_s[pl.ds(1024 + i * 16, 16)]
                stg_t[pl.ds(i * 16, 16)] = stg_t[pl.ds(1024 + i * 16, 16)]
                stg_w[pl.ds(i * 16, 16)] = stg_w[pl.ds(1024 + i * 16, 16)]
                return cc
            lax.fori_loop(0, (l - 1024 + 15) >> 4, sh, 0)
            return (o + 1024, l - 1024)
        return lax.cond(lc >= 1024, spill, lambda a: a, (off, lc))

    _issue_half(0, 0, semA)
    def p5_pair(hp, carry):
        h0 = hp * 2
        _drain_half(0, semA)
        _issue_half(h0 + 1, _HC, semB)
        carry = p5_process(0, carry)
        _drain_half(_HC, semB)

        @pl.when(hp < _NPAIR - 1)
        def _():
            _issue_half(h0 + 2, 0, semA)
        carry = p5_process(_HC, carry)
        return carry
    off, lc = lax.fori_loop(0, _NPAIR, p5_pair, (jnp.int32(0), cnt_own))
    off = _flush(off, lc)
    ngroups = off >> 4

    nch = (off + 1023) >> 10

    def load_sel_chunk(ci):
        cbase = pl.multiple_of(tid * _ROW + ci * 1024, 8)
        d0 = pltpu.async_copy(selS_hbm.at[pl.ds(cbase, 1024)],
                              stg_s.at[pl.ds(0, 1024)], sem)
        d1 = pltpu.async_copy(selT_hbm.at[pl.ds(cbase, 1024)],
                              stg_t.at[pl.ds(0, 1024)], sem)
        d2 = pltpu.async_copy(selW_hbm.at[pl.ds(cbase, 1024)],
                              stg_w.at[pl.ds(0, 1024)], sem)
        d0.wait(); d1.wait(); d2.wait()
    load_sel_chunk(0)

    nzb = (cnt_own + 3) >> 2

    def za0(jb, c):
        pltpu.async_copy(z4, agg_sh.at[pl.ds(nb + jb * 4, 4)], semA)
        return c
    lax.fori_loop(0, nzb, za0, 0)

    def zw0(jb, c):
        pltpu.make_async_copy(z4, agg_sh.at[pl.ds(nb + jb * 4, 4)],
                              semA).wait()
        return c
    lax.fori_loop(0, nzb, zw0, 0)

    def t_loop(t, carry):
        @pl.when(jnp.logical_and(nch == 1, ngroups > 0))
        def _():
            for b in range(2):
                gidx4[pl.ds(b * 16, 16)] = (
                    stg_s[pl.ds(b * 16, 16)] + t * _N)
            pltpu.async_copy(x_hbm.at[gidx4.at[pl.ds(0, 16)]],
                             xrowA.at[pl.ds(0, 16)], sem2)
            pltpu.async_copy(x_hbm.at[gidx4.at[pl.ds(16, 16)]],
                             xrowA.at[pl.ds(16, 16)], sem2)
        plsc.subcore_barrier()

        def cchunk(ci, c):
            @pl.when(jnp.logical_or(ci > 0, nch > 1))
            def _():
                load_sel_chunk(ci)
            ng_here = jnp.minimum(64, ngroups - ci * 64)

            def blk(bi, cc):
                g0 = bi * 2

                @pl.when(jnp.logical_or(bi > 0, nch > 1))
                def _():
                    for b in range(2):
                        gidx4[pl.ds(b * 16, 16)] = (
                            stg_s[pl.ds((g0 + b) * 16, 16)] + t * _N)
                    pltpu.async_copy(x_hbm.at[gidx4.at[pl.ds(0, 16)]],
                                     xrowA.at[pl.ds(0, 16)], sem2)
                    pltpu.async_copy(x_hbm.at[gidx4.at[pl.ds(16, 16)]],
                                     xrowA.at[pl.ds(16, 16)], sem2)
                pltpu.make_async_copy(x_hbm.at[gidx4.at[pl.ds(0, 16)]],
                                      xrowA.at[pl.ds(0, 16)], sem2).wait()
                pltpu.make_async_copy(x_hbm.at[gidx4.at[pl.ds(16, 16)]],
                                      xrowA.at[pl.ds(16, 16)], sem2).wait()
                for b in range(2):
                    nv = stg_w[pl.ds((g0 + b) * 16, 16)]
                    for r in range(16):
                        nrm = nv[r]
                        for q in range(_D // 16):
                            xrowA[b * 16 + r, pl.ds(q * 16, 16)] = (
                                xrowA[b * 16 + r, pl.ds(q * 16, 16)] * nrm)
                    if b == 0:
                        sidx[...] = stg_t[pl.ds(g0 * 16, 16)]
                        pltpu.async_copy(xrowA.at[pl.ds(0, 16)],
                                         agg_sh.at[sidx], semA, add=True)
                    else:
                        sidx2[...] = stg_t[pl.ds((g0 + 1) * 16, 16)]
                        pltpu.async_copy(xrowA.at[pl.ds(16, 16)],
                                         agg_sh.at[sidx2], semA, add=True)
                pltpu.make_async_copy(xrowA.at[pl.ds(0, 16)],
                                      agg_sh.at[sidx], semA).wait()
                pltpu.make_async_copy(xrowA.at[pl.ds(16, 16)],
                                      agg_sh.at[sidx2], semA).wait()
                return cc
            lax.fori_loop(0, ng_here >> 1, blk, 0)
            return c
        lax.fori_loop(0, nch, cchunk, 0)
        plsc.subcore_barrier()

        def consume_row(buf, j):
            w0j = w0cmp[pl.ds(j, 16)][0]

            def kk(kb, acc):
                a0, a1, a2, a3 = acc
                av = buf[pl.ds(kb * 16, 16)]
                for r in range(16):
                    ak = av[r]
                    k = kb * 16 + r
                    a0 = a0 + ak * w1_loc[pl.ds(k * 64, 16)]
                    a1 = a1 + ak * w1_loc[pl.ds(k * 64 + 16, 16)]
                    a2 = a2 + ak * w1_loc[pl.ds(k * 64 + 32, 16)]
                    a3 = a3 + ak * w1_loc[pl.ds(k * 64 + 48, 16)]
                return (a0, a1, a2, a3)
            accs = lax.fori_loop(0, _D // 16, kk, (zf, zf, zf, zf))
            for q in range(4):
                hq = jnp.maximum(accs[q] + b1_loc[pl.ds(q * 16, 16)], 0.0)
                u_loc[pl.ds(t * 64 + q * 16, 16)] = (
                    u_loc[pl.ds(t * 64 + q * 16, 16)] + w0j * hq)

        def cs(jb, c):
            j0 = jb * 2
            have2 = (j0 + 1) < cnt_own
            da = pltpu.async_copy(agg_sh.at[nb + j0], arow, sem2)

            @pl.when(have2)
            def _():
                pltpu.async_copy(agg_sh.at[nb + j0 + 1], arow2, sem2)
            da.wait()
            pltpu.async_copy(z4.at[0], agg_sh.at[nb + j0], semA)
            consume_row(arow, j0)

            @pl.when(have2)
            def _():
                pltpu.make_async_copy(agg_sh.at[nb + j0 + 1], arow2,
                                      sem2).wait()
                pltpu.async_copy(z4.at[0], agg_sh.at[nb + j0 + 1], semA)
                consume_row(arow2, j0 + 1)
            return c
        lax.fori_loop(0, (cnt_own + 1) >> 1, cs, 0)

        def zdrain(j, c):
            pltpu.make_async_copy(z4.at[0], agg_sh.at[nb + j], semA).wait()
            return c
        lax.fori_loop(0, cnt_own, zdrain, 0)
        return carry
    lax.fori_loop(0, _T, t_loop, 0)

    pltpu.sync_copy(u_loc, up_hbm.at[pl.ds(pl.multiple_of(tid * _T * _H, 8), _T * _H)])


def _graph_call(s, d, ew, x2d, w1f, b1):
    return pl.kernel(
        _graph_body,
        out_type=(
            jax.ShapeDtypeStruct((_NT * _T * _H,), jnp.float32),
            jax.ShapeDtypeStruct((_NT * _NPAD,), jnp.float32),
            jax.ShapeDtypeStruct((_NT * _NPAD,), jnp.float32),
            jax.ShapeDtypeStruct((_NT * _ROW,), jnp.int32),
            jax.ShapeDtypeStruct((_NT * _ROW,), jnp.int32),
            jax.ShapeDtypeStruct((_NT * _ROW,), jnp.float32),
        ),
        mesh=plsc.VectorSubcoreMesh(core_axis_name="c", subcore_axis_name="s",
                                    num_cores=1),
        compiler_params=pltpu.CompilerParams(needs_layout_passes=False),
        scratch_types=[
            pltpu.VMEM((_CH,), jnp.int32),
            pltpu.VMEM((_CH,), jnp.int32),
            pltpu.VMEM((_CH,), jnp.float32),
            pltpu.VMEM((_NPAD,), jnp.float32),
            pltpu.VMEM((_NPAD,), jnp.float32),
            pltpu.VMEM((_NPT,), jnp.float32),
            pltpu.VMEM((_NPT,), jnp.float32),
            pltpu.VMEM((_NPT,), jnp.float32),
            pltpu.VMEM((_NPT + 16,), jnp.float32),
            pltpu.VMEM((4, _D), jnp.float32),
            pltpu.VMEM((_STG,), jnp.int32),
            pltpu.VMEM((_STG,), jnp.int32),
            pltpu.VMEM((_STG,), jnp.float32),
            pltpu.VMEM((_D * _H,), jnp.float32),
            pltpu.VMEM((_H,), jnp.float32),
            pltpu.VMEM((32,), jnp.int32),
            pltpu.VMEM((16,), jnp.int32),
            pltpu.VMEM((16,), jnp.int32),
            pltpu.VMEM((32, _D), jnp.float32),
            pltpu.VMEM((_D,), jnp.float32),
            pltpu.VMEM((_D,), jnp.float32),
            pltpu.VMEM((_T * _H,), jnp.float32),
            pltpu.VMEM_SHARED((_NPAD,), jnp.float32),
            pltpu.VMEM_SHARED((_NPAD,), jnp.float32),
            pltpu.VMEM_SHARED((_AGG_ROWS, _D), jnp.float32),
            pltpu.SemaphoreType.DMA,
            pltpu.SemaphoreType.DMA,
            pltpu.SemaphoreType.DMA,
            pltpu.SemaphoreType.DMA,
        ],
    )(s, d, ew, x2d, w1f, b1)


def _head_body(m_ref, up_ref, w2_ref, b2_ref, wih_ref, whh_ref,
               bih_ref, bhh_ref, hwt_ref, hb_ref, out_ref):
    u = jnp.dot(m_ref[...], up_ref[...], preferred_element_type=jnp.float32)
    seq = jnp.maximum(
        jnp.dot(u, w2_ref[...], preferred_element_type=jnp.float32)
        + b2_ref[...][None, :], 0.0)
    h = jnp.zeros((1, _H), jnp.float32)
    for t in range(_T):
        xt = seq[t:t + 1, :]
        gi = jnp.dot(xt, wih_ref[...],
                     preferred_element_type=jnp.float32) + bih_ref[...][None, :]
        gh = jnp.dot(h, whh_ref[...],
                     preferred_element_type=jnp.float32) + bhh_ref[...][None, :]
        r = jax.nn.sigmoid(gi[:, :_H] + gh[:, :_H])
        z = jax.nn.sigmoid(gi[:, _H:2 * _H] + gh[:, _H:2 * _H])
        n = jnp.tanh(gi[:, 2 * _H:] + r * gh[:, 2 * _H:])
        h = (1.0 - z) * n + z * h
    logits = jnp.dot(h, hwt_ref[...],
                     preferred_element_type=jnp.float32) + hb_ref[...][None, :]
    out_ref[...] = jax.nn.sigmoid(logits)


def _head_call(m, up, W2, b2, wihT, whhT, b_ih, b_hh, hWT, heads_b):
    return pl.pallas_call(
        _head_body,
        out_shape=jax.ShapeDtypeStruct((1, 8), jnp.float32),
    )(m, up, W2, b2, wihT, whhT, b_ih, b_hh, hWT, heads_b)


def kernel(x, edge_index, edge_weight, W1, b1, W2, b2, W_ih, W_hh,
           b_ih, b_hh, heads_W, heads_b):
    s = edge_index[0].astype(jnp.int32)
    d = edge_index[1].astype(jnp.int32)
    x2d = x.reshape(_T * _N, _D)
    outs = _graph_call(s, d, edge_weight.astype(jnp.float32), x2d,
                       W1.reshape(-1), b1)
    up = outs[0].reshape(_NT * _T, _H)
    m = jnp.tile(jnp.eye(_T, dtype=jnp.float32), (1, _NT))
    return _head_call(m, up, W2, b2, W_ih.T, W_hh.T, b_ih, b_hh,
                      heads_W.T, heads_b)
